# baseline (speedup 1.0000x reference)
import functools
import numpy as np
import jax
import jax.numpy as jnp
from jax import lax
from jax.experimental import pallas as pl
from jax.experimental.pallas import tpu as pltpu
from jax.experimental.pallas import tpu_sc as plsc

_N = 10000
_E = 160000
_D = 256
_H = 8
_DH = 32
_F32 = jnp.float32
_BF16 = jnp.bfloat16


def _qkv_body(v_ref, w_ref, q_ref, k_ref, vv_ref):
    y = jnp.dot(v_ref[...], w_ref[...], preferred_element_type=_F32)
    y = y.astype(_BF16)
    q_ref[...] = y[:, :_D]
    k_ref[...] = y[:, _D:2 * _D]
    vv_ref[...] = y[:, 2 * _D:]


def _qkv_call(v, wqkv):
    nb = 400
    grid = (_N // nb,)
    return pl.pallas_call(
        _qkv_body,
        grid=grid,
        in_specs=[
            pl.BlockSpec((nb, _D), lambda i: (i, 0)),
            pl.BlockSpec((_D, 3 * _D), lambda i: (0, 0)),
        ],
        out_specs=[
            pl.BlockSpec((nb, _D), lambda i: (i, 0)),
            pl.BlockSpec((nb, _D), lambda i: (i, 0)),
            pl.BlockSpec((nb, _D), lambda i: (i, 0)),
        ],
        out_shape=[jax.ShapeDtypeStruct((_N, _D), _BF16)] * 3,
    )(v, wqkv)


def _edge_a_body(e_ref, ks_ref, qd_ref, vs_ref, we_ref, woe_ref, boe_ref,
                 smask_ref, bmask_ref, e1_ref, ev_ref, s16_ref, ssum_ref,
                 ssq_ref):
    eb = e_ref[...]
    pe = jnp.dot(eb.astype(_BF16), we_ref[...], preferred_element_type=_F32)
    kq = ks_ref[...].astype(_F32) * qd_ref[...].astype(_F32)
    score = kq * pe * np.float32(1.0 / np.sqrt(_DH))
    shead = jnp.dot(score, smask_ref[...], preferred_element_type=_F32)
    sexp = jnp.exp(jnp.clip(shead, -5.0, 5.0))
    e1 = eb + jnp.dot(score.astype(_BF16), woe_ref[...],
                      preferred_element_type=_F32) + boe_ref[...]
    e1_ref[...] = e1
    ev_ref[...] = vs_ref[...].astype(_F32) * jnp.dot(
        sexp, bmask_ref[...], preferred_element_type=_F32)
    s16_ref[...] = jnp.concatenate(
        [sexp, jnp.zeros((sexp.shape[0], 8), _F32)], axis=1)

    @pl.when(pl.program_id(0) == 0)
    def _():
        ssum_ref[...] = jnp.zeros_like(ssum_ref)
        ssq_ref[...] = jnp.zeros_like(ssq_ref)

    ssum_ref[...] += jnp.sum(e1, axis=0, keepdims=True)
    ssq_ref[...] += jnp.sum(e1 * e1, axis=0, keepdims=True)


def _edge_a_call(e, ksrc, qdst, vsrc, we, woe, boe, smask, bmask):
    eb = 1000
    grid = (_E // eb,)
    big = pl.BlockSpec((eb, _D), lambda i: (i, 0))
    return pl.pallas_call(
        _edge_a_body,
        grid=grid,
        in_specs=[
            big, big, big, big,
            pl.BlockSpec((_D, _D), lambda i: (0, 0)),
            pl.BlockSpec((_D, _D), lambda i: (0, 0)),
            pl.BlockSpec((1, _D), lambda i: (0, 0)),
            pl.BlockSpec((_D, _H), lambda i: (0, 0)),
            pl.BlockSpec((_H, _D), lambda i: (0, 0)),
        ],
        out_specs=[
            big, big,
            pl.BlockSpec((eb, 16), lambda i: (i, 0)),
            pl.BlockSpec((1, _D), lambda i: (0, 0)),
            pl.BlockSpec((1, _D), lambda i: (0, 0)),
        ],
        out_shape=[
            jax.ShapeDtypeStruct((_E, _D), _F32),
            jax.ShapeDtypeStruct((_E, _D), _F32),
            jax.ShapeDtypeStruct((_E, 16), _F32),
            jax.ShapeDtypeStruct((1, _D), _F32),
            jax.ShapeDtypeStruct((1, _D), _F32),
        ],
    )(e, ksrc, qdst, vsrc, we, woe, boe, smask, bmask)


def _vatt_body(wv_ref, z_ref, v_ref, wov_ref, bov_ref, bmz_ref, v1_ref,
               ssum_ref, ssq_ref):
    zb = jnp.dot(z_ref[...], bmz_ref[...], preferred_element_type=_F32)
    vatt = wv_ref[...] / (zb + 1e-6)
    v1 = v_ref[...] + jnp.dot(vatt.astype(_BF16), wov_ref[...],
                              preferred_element_type=_F32) + bov_ref[...]
    v1_ref[...] = v1

    @pl.when(pl.program_id(0) == 0)
    def _():
        ssum_ref[...] = jnp.zeros_like(ssum_ref)
        ssq_ref[...] = jnp.zeros_like(ssq_ref)

    ssum_ref[...] += jnp.sum(v1, axis=0, keepdims=True)
    ssq_ref[...] += jnp.sum(v1 * v1, axis=0, keepdims=True)


def _vatt_call(wv, z16, v, wov, bov, bmz):
    nb = 400
    grid = (_N // nb,)
    return pl.pallas_call(
        _vatt_body,
        grid=grid,
        in_specs=[
            pl.BlockSpec((nb, _D), lambda i: (i, 0)),
            pl.BlockSpec((nb, 16), lambda i: (i, 0)),
            pl.BlockSpec((nb, _D), lambda i: (i, 0)),
            pl.BlockSpec((_D, _D), lambda i: (0, 0)),
            pl.BlockSpec((1, _D), lambda i: (0, 0)),
            pl.BlockSpec((16, _D), lambda i: (0, 0)),
        ],
        out_specs=[
            pl.BlockSpec((nb, _D), lambda i: (i, 0)),
            pl.BlockSpec((1, _D), lambda i: (0, 0)),
            pl.BlockSpec((1, _D), lambda i: (0, 0)),
        ],
        out_shape=[
            jax.ShapeDtypeStruct((_N, _D), _F32),
            jax.ShapeDtypeStruct((1, _D), _F32),
            jax.ShapeDtypeStruct((1, _D), _F32),
        ],
    )(wv, z16, v, wov, bov, bmz)


def _bnffn_body(x_ref, m_ref, r_ref, g_ref, b_ref, w1_ref, b1_ref, w2_ref,
                b2_ref, y_ref, ssum_ref, ssq_ref):
    xn = (x_ref[...] - m_ref[...]) * r_ref[...] * g_ref[...] + b_ref[...]
    h = jnp.maximum(
        jnp.dot(xn.astype(_BF16), w1_ref[...],
                preferred_element_type=_F32) + b1_ref[...], 0.0)
    y = xn + jnp.dot(h.astype(_BF16), w2_ref[...],
                     preferred_element_type=_F32) + b2_ref[...]
    y_ref[...] = y

    @pl.when(pl.program_id(0) == 0)
    def _():
        ssum_ref[...] = jnp.zeros_like(ssum_ref)
        ssq_ref[...] = jnp.zeros_like(ssq_ref)

    ssum_ref[...] += jnp.sum(y, axis=0, keepdims=True)
    ssq_ref[...] += jnp.sum(y * y, axis=0, keepdims=True)


def _bnffn_call(x, m, r, g, b, w1, b1, w2, b2, rows, rb):
    grid = (rows // rb,)
    vec = pl.BlockSpec((1, _D), lambda i: (0, 0))
    return pl.pallas_call(
        _bnffn_body,
        grid=grid,
        in_specs=[
            pl.BlockSpec((rb, _D), lambda i: (i, 0)),
            vec, vec, vec, vec,
            pl.BlockSpec((_D, 2 * _D), lambda i: (0, 0)),
            pl.BlockSpec((1, 2 * _D), lambda i: (0, 0)),
            pl.BlockSpec((2 * _D, _D), lambda i: (0, 0)),
            vec,
        ],
        out_specs=[
            pl.BlockSpec((rb, _D), lambda i: (i, 0)),
            pl.BlockSpec((1, _D), lambda i: (0, 0)),
            pl.BlockSpec((1, _D), lambda i: (0, 0)),
        ],
        out_shape=[
            jax.ShapeDtypeStruct((rows, _D), _F32),
            jax.ShapeDtypeStruct((1, _D), _F32),
            jax.ShapeDtypeStruct((1, _D), _F32),
        ],
    )(x, m, r, g, b, w1, b1, w2, b2)


def _bnapply_body(x_ref, m_ref, r_ref, g_ref, b_ref, y_ref):
    y_ref[...] = (x_ref[...] - m_ref[...]) * r_ref[...] * g_ref[...] \
        + b_ref[...]


def _bnapply_call(x, m, r, g, b, rows, rb):
    grid = (rows // rb,)
    vec = pl.BlockSpec((1, _D), lambda i: (0, 0))
    return pl.pallas_call(
        _bnapply_body,
        grid=grid,
        in_specs=[pl.BlockSpec((rb, _D), lambda i: (i, 0)), vec, vec, vec,
                  vec],
        out_specs=pl.BlockSpec((rb, _D), lambda i: (i, 0)),
        out_shape=jax.ShapeDtypeStruct((rows, _D), _F32),
    )(x, m, r, g, b)


_NC = 2
_NS = 16
_NW = _NC * _NS
_GC = 40
_DP = 128
_GPW = _E // _NW
_SC = 80
_SPW = _E // _NS
_NP = 10240
_RPS = _NP // _NS
_ZH = _NP // _NC
_ZP = 5248
_ZRS = _ZP // _NS


def _gather3_build():
    mesh = plsc.VectorSubcoreMesh(core_axis_name="c", subcore_axis_name="s", num_cores=_NC, num_subcores=_NS)

    @functools.partial(
        pl.kernel,
        out_type=(
            jax.ShapeDtypeStruct((_E, _DP), _F32),
            jax.ShapeDtypeStruct((_E, _DP), _F32),
            jax.ShapeDtypeStruct((_E, _DP), _F32),
        ),
        mesh=mesh,
        scratch_types=[
            pltpu.VMEM((_GC,), jnp.int32),
            pltpu.VMEM((_GC,), jnp.int32),
            pltpu.VMEM((_GC, _DP), _F32),
            pltpu.VMEM((_GC, _DP), _F32),
            pltpu.VMEM((_GC, _DP), _F32),
            pltpu.SemaphoreType.DMA,
        ],
    )
    def gather3(ktab, qtab, vtab, src, dst, ok, oq, ov, src_v, dst_v, bk, bq,
                bv, sem):
        wid = lax.axis_index("s") * _NC + lax.axis_index("c")
        base = wid * _GPW

        def body(j, carry):
            off = base + j * _GC
            pltpu.sync_copy(src.at[pl.ds(off, _GC)], src_v)
            pltpu.sync_copy(dst.at[pl.ds(off, _GC)], dst_v)
            ck = pltpu.async_copy(ktab.at[src_v], bk, sem)
            cq = pltpu.async_copy(qtab.at[dst_v], bq, sem)
            cv = pltpu.async_copy(vtab.at[src_v], bv, sem)
            ck.wait()
            cq.wait()
            cv.wait()
            pltpu.sync_copy(bk, ok.at[pl.ds(off, _GC)])
            pltpu.sync_copy(bq, oq.at[pl.ds(off, _GC)])
            pltpu.sync_copy(bv, ov.at[pl.ds(off, _GC)])
            return carry

        lax.fori_loop(0, _GPW // _GC, body, 0)

    return gather3


def _scatter_build():
    mesh = plsc.VectorSubcoreMesh(core_axis_name="c", subcore_axis_name="s", num_cores=_NC, num_subcores=_NS)

    @functools.partial(
        pl.kernel,
        out_type=(
            jax.ShapeDtypeStruct((_NP, _D), _F32),
            jax.ShapeDtypeStruct((_NC * _ZP, 128), _F32),
        ),
        mesh=mesh,
        scratch_types=[
            pltpu.VMEM((_SC,), jnp.int32),
            pltpu.VMEM((_SC,), jnp.int32),
            pltpu.VMEM((_SC, 128), _F32),
            pltpu.VMEM((_SC, 128), _F32),
            pltpu.VMEM((_SC * 16,), _F32),
            pltpu.VMEM_SHARED((_NP, 128), _F32),
        ],
    )
    def scatter(ev, s16, dst, zrows, owv, oz, dst_v, dstc_v, evb, zb, sb,
                acc):
        cid = lax.axis_index("c")
        sid = lax.axis_index("s")
        col0 = cid * 128
        node0 = cid * _ZH

        pltpu.sync_copy(zrows, acc.at[pl.ds(sid * _RPS, _RPS)])
        plsc.subcore_barrier()

        def body_ev(j, carry):
            off = sid * _SPW + j * _SC
            pltpu.sync_copy(dst.at[pl.ds(off, _SC)], dst_v)
            pltpu.sync_copy(ev.at[pl.ds(off, _SC), pl.ds(col0, 128)], evb)
            pltpu.sync_copy(evb, acc.at[dst_v], add=True)
            return carry

        lax.fori_loop(0, _SPW // _SC, body_ev, 0)
        plsc.subcore_barrier()
        r0 = sid * _RPS
        pltpu.sync_copy(acc.at[pl.ds(r0, _RPS)],
                        owv.at[pl.ds(r0, _RPS), pl.ds(col0, 128)])
        plsc.subcore_barrier()

        pltpu.sync_copy(zrows.at[pl.ds(0, _ZRS)],
                        acc.at[pl.ds(sid * _ZRS, _ZRS)])
        plsc.subcore_barrier()

        pltpu.sync_copy(zrows.at[pl.ds(0, _SC)], zb)

        def body_z(j, carry):
            off = sid * _SPW + j * _SC
            pltpu.sync_copy(dst.at[pl.ds(off, _SC)], dst_v)
            pltpu.sync_copy(s16.at[pl.ds(off * 16, _SC * 16)], sb)
            for t in range(_SC):
                zb[t, pl.ds(0, 16)] = sb[pl.ds(t * 16, 16)]
            for t in range(_SC // 16):
                iv = dst_v[pl.ds(t * 16, 16)]
                rel = iv - node0
                good = (rel >= 0) & (rel < _ZH)
                dstc_v[pl.ds(t * 16, 16)] = jnp.where(good, rel, _ZH)
            pltpu.sync_copy(zb, acc.at[dstc_v], add=True)
            return carry

        lax.fori_loop(0, _SPW // _SC, body_z, 0)
        plsc.subcore_barrier()
        rz = sid * _ZRS
        pltpu.sync_copy(acc.at[pl.ds(rz, _ZRS)],
                        oz.at[pl.ds(cid * _ZP + rz, _ZRS)])

    return scatter


_GATHER3 = None
_SCATTER = None


def _gather3_run(k, q, vv, src, dst):
    global _GATHER3
    if _GATHER3 is None:
        _GATHER3 = _gather3_build()
    return _GATHER3(k, q, vv, src, dst)


def _scatter_run(ev, s16, dst, zrows):
    global _SCATTER
    if _SCATTER is None:
        _SCATTER = _scatter_build()
    return _SCATTER(ev, s16, dst, zrows)


_SMASK = (np.arange(_D)[:, None] // _DH ==
          np.arange(_H)[None, :]).astype(np.float32)
_BMASK = (np.arange(_D)[None, :] // _DH ==
          np.arange(_H)[:, None]).astype(np.float32)
_BMZ = np.concatenate([_BMASK, np.zeros((8, _D), np.float32)], axis=0)


def _row(x):
    return x.reshape(1, -1)


def kernel(v, e, edge_index, WQ, WK, WV, We, WOv, bOv, WOe, bOe, W1v, b1v,
           W2v, b2v, W1e, b1e, W2e, b2e, g1v, be1v, g1e, be1e, g2v, be2v,
           g2e, be2e):
    src = edge_index[0]
    dst = edge_index[1]
    wqkv = jnp.concatenate([WQ, WK, WV], axis=1)
    we16 = We.astype(_BF16)
    woe16 = WOe.astype(_BF16)
    wov16 = WOv.astype(_BF16)

    q, k, vv = _qkv_call(v, wqkv)

    def _pack(x):
        return lax.bitcast_convert_type(x.reshape(_N, _DP, 2), _F32)

    def _unpack(x):
        return lax.bitcast_convert_type(x, _BF16).reshape(_E, _D)

    ksp, qdp, vsp = _gather3_run(_pack(k), _pack(q), _pack(vv), src, dst)
    ksrc = _unpack(ksp)
    qdst = _unpack(qdp)
    vsrc = _unpack(vsp)

    e1, ev, s16, s1, q1 = _edge_a_call(e, ksrc, qdst, vsrc, we16, woe16,
                                       _row(bOe), _SMASK, _BMASK)

    zrows = jnp.zeros((_RPS, 128), _F32)
    wv_pad, oz = _scatter_run(ev, s16.reshape(-1), dst, zrows)
    wv = wv_pad[:_N]
    z16 = jnp.concatenate(
        [oz[:_ZH, :16], oz[_ZP:_ZP + _N - _ZH, :16]], axis=0)

    m1 = s1 / _E
    r1 = lax.rsqrt(q1 / _E - m1 * m1 + 1e-5)
    e2, s2, q2 = _bnffn_call(e1, m1, r1, _row(g1e), _row(be1e), W1e.astype(_BF16),
                             _row(b1e), W2e, _row(b2e), _E, 1000)
    m2 = s2 / _E
    r2 = lax.rsqrt(q2 / _E - m2 * m2 + 1e-5)
    out_e = _bnapply_call(e2, m2, r2, _row(g2e), _row(be2e), _E, 1000)

    v1, sv1, qv1 = _vatt_call(wv, z16, v, wov16, _row(bOv), _BMZ)
    mv1 = sv1 / _N
    rv1 = lax.rsqrt(qv1 / _N - mv1 * mv1 + 1e-5)
    v2, sv2, qv2 = _bnffn_call(v1, mv1, rv1, _row(g1v), _row(be1v), W1v.astype(_BF16),
                               _row(b1v), W2v.astype(_BF16), _row(b2v), _N, 400)
    mv2 = sv2 / _N
    rv2 = lax.rsqrt(qv2 / _N - mv2 * mv2 + 1e-5)
    out_v = _bnapply_call(v2, mv2, rv2, _row(g2v), _row(be2v), _N, 400)

    return (out_v, out_e)


# speedup vs baseline: 2.5649x; 2.5649x over previous
import functools
import numpy as np
import jax
import jax.numpy as jnp
from jax import lax
from jax.experimental import pallas as pl
from jax.experimental.pallas import tpu as pltpu
from jax.experimental.pallas import tpu_sc as plsc

_N = 10000
_E = 160000
_D = 256
_H = 8
_DH = 32
_F32 = jnp.float32
_BF16 = jnp.bfloat16


def _pack_cols(y):
    lo = lax.bitcast_convert_type(y[:, :128].astype(_BF16),
                                  jnp.uint16).astype(jnp.uint32)
    hi = lax.bitcast_convert_type(y[:, 128:].astype(_BF16),
                                  jnp.uint16).astype(jnp.uint32)
    return lax.bitcast_convert_type(lo | (hi << 16), _F32)


def _unpack_cols(x):
    xi = lax.bitcast_convert_type(x, jnp.uint32)
    lo = lax.bitcast_convert_type(xi << 16, _F32)
    hi = lax.bitcast_convert_type(xi & jnp.uint32(0xFFFF0000), _F32)
    return jnp.concatenate([lo, hi], axis=1)


def _qkv_body(v_ref, w_ref, q_ref, k_ref, vv_ref):
    y = jnp.dot(v_ref[...], w_ref[...], preferred_element_type=_F32)
    q_ref[...] = _pack_cols(y[:, :_D])
    k_ref[...] = _pack_cols(y[:, _D:2 * _D])
    vv_ref[...] = _pack_cols(y[:, 2 * _D:])


def _qkv_call(v, wqkv):
    nb = 400
    grid = (_N // nb,)
    return pl.pallas_call(
        _qkv_body,
        grid=grid,
        in_specs=[
            pl.BlockSpec((nb, _D), lambda i: (i, 0)),
            pl.BlockSpec((_D, 3 * _D), lambda i: (0, 0)),
        ],
        out_specs=[
            pl.BlockSpec((nb, 128), lambda i: (i, 0)),
            pl.BlockSpec((nb, 128), lambda i: (i, 0)),
            pl.BlockSpec((nb, 128), lambda i: (i, 0)),
        ],
        out_shape=[jax.ShapeDtypeStruct((_N, 128), _F32)] * 3,
    )(v, wqkv)


def _edge_a_body(e_ref, ks_ref, qd_ref, vs_ref, we_ref, woe_ref, boe_ref,
                 smask_ref, bmask_ref, e1_ref, ev_ref, s16_ref, ssum_ref,
                 ssq_ref):
    eb = e_ref[...]
    pe = jnp.dot(eb.astype(_BF16), we_ref[...], preferred_element_type=_F32)
    kq = _unpack_cols(ks_ref[...]) * _unpack_cols(qd_ref[...])
    score = kq * pe * np.float32(1.0 / np.sqrt(_DH))
    shead = jnp.dot(score, smask_ref[...], preferred_element_type=_F32)
    sexp = jnp.exp(jnp.clip(shead, -5.0, 5.0))
    e1 = eb + jnp.dot(score.astype(_BF16), woe_ref[...],
                      preferred_element_type=_F32) + boe_ref[...]
    e1_ref[...] = e1
    ev_ref[...] = _unpack_cols(vs_ref[...]) * jnp.dot(
        sexp, bmask_ref[...], preferred_element_type=_F32)
    s16_ref[...] = jnp.concatenate(
        [sexp, jnp.zeros((sexp.shape[0], 8), _F32)], axis=1)

    @pl.when(pl.program_id(0) == 0)
    def _():
        ssum_ref[...] = jnp.zeros_like(ssum_ref)
        ssq_ref[...] = jnp.zeros_like(ssq_ref)

    ssum_ref[...] += jnp.sum(e1, axis=0, keepdims=True)
    ssq_ref[...] += jnp.sum(e1 * e1, axis=0, keepdims=True)


def _edge_a_call(e, ksrc, qdst, vsrc, we, woe, boe, smask, bmask):
    eb = 1000
    grid = (_E // eb,)
    big = pl.BlockSpec((eb, _D), lambda i: (i, 0))
    pk = pl.BlockSpec((eb, 128), lambda i: (i, 0))
    return pl.pallas_call(
        _edge_a_body,
        grid=grid,
        in_specs=[
            big, pk, pk, pk,
            pl.BlockSpec((_D, _D), lambda i: (0, 0)),
            pl.BlockSpec((_D, _D), lambda i: (0, 0)),
            pl.BlockSpec((1, _D), lambda i: (0, 0)),
            pl.BlockSpec((_D, _H), lambda i: (0, 0)),
            pl.BlockSpec((_H, _D), lambda i: (0, 0)),
        ],
        out_specs=[
            big, big,
            pl.BlockSpec((eb, 16), lambda i: (i, 0)),
            pl.BlockSpec((1, _D), lambda i: (0, 0)),
            pl.BlockSpec((1, _D), lambda i: (0, 0)),
        ],
        out_shape=[
            jax.ShapeDtypeStruct((_E, _D), _F32),
            jax.ShapeDtypeStruct((_E, _D), _F32),
            jax.ShapeDtypeStruct((_E, 16), _F32),
            jax.ShapeDtypeStruct((1, _D), _F32),
            jax.ShapeDtypeStruct((1, _D), _F32),
        ],
    )(e, ksrc, qdst, vsrc, we, woe, boe, smask, bmask)


def _vatt_body(wv_ref, z_ref, v_ref, wov_ref, bov_ref, bmz_ref, v1_ref,
               ssum_ref, ssq_ref):
    zb = jnp.dot(z_ref[...], bmz_ref[...], preferred_element_type=_F32)
    vatt = wv_ref[...] / (zb + 1e-6)
    v1 = v_ref[...] + jnp.dot(vatt.astype(_BF16), wov_ref[...],
                              preferred_element_type=_F32) + bov_ref[...]
    v1_ref[...] = v1

    @pl.when(pl.program_id(0) == 0)
    def _():
        ssum_ref[...] = jnp.zeros_like(ssum_ref)
        ssq_ref[...] = jnp.zeros_like(ssq_ref)

    ssum_ref[...] += jnp.sum(v1, axis=0, keepdims=True)
    ssq_ref[...] += jnp.sum(v1 * v1, axis=0, keepdims=True)


def _vatt_call(wv, z16, v, wov, bov, bmz):
    nb = 400
    grid = (_N // nb,)
    return pl.pallas_call(
        _vatt_body,
        grid=grid,
        in_specs=[
            pl.BlockSpec((nb, _D), lambda i: (i, 0)),
            pl.BlockSpec((nb, 16), lambda i: (i, 0)),
            pl.BlockSpec((nb, _D), lambda i: (i, 0)),
            pl.BlockSpec((_D, _D), lambda i: (0, 0)),
            pl.BlockSpec((1, _D), lambda i: (0, 0)),
            pl.BlockSpec((16, _D), lambda i: (0, 0)),
        ],
        out_specs=[
            pl.BlockSpec((nb, _D), lambda i: (i, 0)),
            pl.BlockSpec((1, _D), lambda i: (0, 0)),
            pl.BlockSpec((1, _D), lambda i: (0, 0)),
        ],
        out_shape=[
            jax.ShapeDtypeStruct((_N, _D), _F32),
            jax.ShapeDtypeStruct((1, _D), _F32),
            jax.ShapeDtypeStruct((1, _D), _F32),
        ],
    )(wv, z16, v, wov, bov, bmz)


def _bnffn_body(x_ref, m_ref, r_ref, g_ref, b_ref, w1_ref, b1_ref, w2_ref,
                b2_ref, y_ref, ssum_ref, ssq_ref):
    xn = (x_ref[...] - m_ref[...]) * r_ref[...] * g_ref[...] + b_ref[...]
    h = jnp.maximum(
        jnp.dot(xn.astype(_BF16), w1_ref[...],
                preferred_element_type=_F32) + b1_ref[...], 0.0)
    y = xn + jnp.dot(h.astype(_BF16), w2_ref[...],
                     preferred_element_type=_F32) + b2_ref[...]
    y_ref[...] = y

    @pl.when(pl.program_id(0) == 0)
    def _():
        ssum_ref[...] = jnp.zeros_like(ssum_ref)
        ssq_ref[...] = jnp.zeros_like(ssq_ref)

    ssum_ref[...] += jnp.sum(y, axis=0, keepdims=True)
    ssq_ref[...] += jnp.sum(y * y, axis=0, keepdims=True)


def _bnffn_call(x, m, r, g, b, w1, b1, w2, b2, rows, rb):
    grid = (rows // rb,)
    vec = pl.BlockSpec((1, _D), lambda i: (0, 0))
    return pl.pallas_call(
        _bnffn_body,
        grid=grid,
        in_specs=[
            pl.BlockSpec((rb, _D), lambda i: (i, 0)),
            vec, vec, vec, vec,
            pl.BlockSpec((_D, 2 * _D), lambda i: (0, 0)),
            pl.BlockSpec((1, 2 * _D), lambda i: (0, 0)),
            pl.BlockSpec((2 * _D, _D), lambda i: (0, 0)),
            vec,
        ],
        out_specs=[
            pl.BlockSpec((rb, _D), lambda i: (i, 0)),
            pl.BlockSpec((1, _D), lambda i: (0, 0)),
            pl.BlockSpec((1, _D), lambda i: (0, 0)),
        ],
        out_shape=[
            jax.ShapeDtypeStruct((rows, _D), _F32),
            jax.ShapeDtypeStruct((1, _D), _F32),
            jax.ShapeDtypeStruct((1, _D), _F32),
        ],
    )(x, m, r, g, b, w1, b1, w2, b2)


def _bnapply_body(x_ref, m_ref, r_ref, g_ref, b_ref, y_ref):
    y_ref[...] = (x_ref[...] - m_ref[...]) * r_ref[...] * g_ref[...] \
        + b_ref[...]


def _bnapply_call(x, m, r, g, b, rows, rb):
    grid = (rows // rb,)
    vec = pl.BlockSpec((1, _D), lambda i: (0, 0))
    return pl.pallas_call(
        _bnapply_body,
        grid=grid,
        in_specs=[pl.BlockSpec((rb, _D), lambda i: (i, 0)), vec, vec, vec,
                  vec],
        out_specs=pl.BlockSpec((rb, _D), lambda i: (i, 0)),
        out_shape=jax.ShapeDtypeStruct((rows, _D), _F32),
    )(x, m, r, g, b)


_NC = 2
_NS = 16
_NW = _NC * _NS
_GC = 40
_DP = 128
_GPW = _E // _NW
_SC = 80
_SPW = _E // _NS
_NP = 10240
_RPS = _NP // _NS
_ZH = _NP // _NC
_ZP = 5248
_ZRS = _ZP // _NS


def _gather3_build():
    mesh = plsc.VectorSubcoreMesh(core_axis_name="c", subcore_axis_name="s", num_cores=_NC, num_subcores=_NS)

    @functools.partial(
        pl.kernel,
        out_type=(
            jax.ShapeDtypeStruct((_E, _DP), _F32),
            jax.ShapeDtypeStruct((_E, _DP), _F32),
            jax.ShapeDtypeStruct((_E, _DP), _F32),
        ),
        mesh=mesh,
        scratch_types=[
            pltpu.VMEM((_GC,), jnp.int32),
            pltpu.VMEM((_GC,), jnp.int32),
            pltpu.VMEM((_GC, _DP), _F32),
            pltpu.VMEM((_GC, _DP), _F32),
            pltpu.VMEM((_GC, _DP), _F32),
            pltpu.SemaphoreType.DMA,
        ],
    )
    def gather3(ktab, qtab, vtab, src, dst, ok, oq, ov, src_v, dst_v, bk, bq,
                bv, sem):
        wid = lax.axis_index("s") * _NC + lax.axis_index("c")
        base = wid * _GPW

        def body(j, carry):
            off = base + j * _GC
            pltpu.sync_copy(src.at[pl.ds(off, _GC)], src_v)
            pltpu.sync_copy(dst.at[pl.ds(off, _GC)], dst_v)
            ck = pltpu.async_copy(ktab.at[src_v], bk, sem)
            cq = pltpu.async_copy(qtab.at[dst_v], bq, sem)
            cv = pltpu.async_copy(vtab.at[src_v], bv, sem)
            ck.wait()
            cq.wait()
            cv.wait()
            pltpu.sync_copy(bk, ok.at[pl.ds(off, _GC)])
            pltpu.sync_copy(bq, oq.at[pl.ds(off, _GC)])
            pltpu.sync_copy(bv, ov.at[pl.ds(off, _GC)])
            return carry

        lax.fori_loop(0, _GPW // _GC, body, 0)

    return gather3


def _scatter_build():
    mesh = plsc.VectorSubcoreMesh(core_axis_name="c", subcore_axis_name="s", num_cores=_NC, num_subcores=_NS)

    @functools.partial(
        pl.kernel,
        out_type=(
            jax.ShapeDtypeStruct((_NP, _D), _F32),
            jax.ShapeDtypeStruct((_NC * _ZP, 128), _F32),
        ),
        mesh=mesh,
        scratch_types=[
            pltpu.VMEM((_SC,), jnp.int32),
            pltpu.VMEM((_SC,), jnp.int32),
            pltpu.VMEM((_SC, 128), _F32),
            pltpu.VMEM((_SC, 128), _F32),
            pltpu.VMEM((_SC * 16,), _F32),
            pltpu.VMEM_SHARED((_NP, 128), _F32),
        ],
    )
    def scatter(ev, s16, dst, zrows, owv, oz, dst_v, dstc_v, evb, zb, sb,
                acc):
        cid = lax.axis_index("c")
        sid = lax.axis_index("s")
        col0 = cid * 128
        node0 = cid * _ZH

        pltpu.sync_copy(zrows, acc.at[pl.ds(sid * _RPS, _RPS)])
        plsc.subcore_barrier()

        def body_ev(j, carry):
            off = sid * _SPW + j * _SC
            pltpu.sync_copy(dst.at[pl.ds(off, _SC)], dst_v)
            pltpu.sync_copy(ev.at[pl.ds(off, _SC), pl.ds(col0, 128)], evb)
            pltpu.sync_copy(evb, acc.at[dst_v], add=True)
            return carry

        lax.fori_loop(0, _SPW // _SC, body_ev, 0)
        plsc.subcore_barrier()
        r0 = sid * _RPS
        pltpu.sync_copy(acc.at[pl.ds(r0, _RPS)],
                        owv.at[pl.ds(r0, _RPS), pl.ds(col0, 128)])
        plsc.subcore_barrier()

        pltpu.sync_copy(zrows.at[pl.ds(0, _ZRS)],
                        acc.at[pl.ds(sid * _ZRS, _ZRS)])
        plsc.subcore_barrier()

        pltpu.sync_copy(zrows.at[pl.ds(0, _SC)], zb)

        def body_z(j, carry):
            off = sid * _SPW + j * _SC
            pltpu.sync_copy(dst.at[pl.ds(off, _SC)], dst_v)
            pltpu.sync_copy(s16.at[pl.ds(off * 16, _SC * 16)], sb)
            for t in range(_SC):
                zb[t, pl.ds(0, 16)] = sb[pl.ds(t * 16, 16)]
            for t in range(_SC // 16):
                iv = dst_v[pl.ds(t * 16, 16)]
                rel = iv - node0
                good = (rel >= 0) & (rel < _ZH)
                dstc_v[pl.ds(t * 16, 16)] = jnp.where(good, rel, _ZH)
            pltpu.sync_copy(zb, acc.at[dstc_v], add=True)
            return carry

        lax.fori_loop(0, _SPW // _SC, body_z, 0)
        plsc.subcore_barrier()
        rz = sid * _ZRS
        pltpu.sync_copy(acc.at[pl.ds(rz, _ZRS)],
                        oz.at[pl.ds(cid * _ZP + rz, _ZRS)])

    return scatter


_GATHER3 = None
_SCATTER = None


def _gather3_run(k, q, vv, src, dst):
    global _GATHER3
    if _GATHER3 is None:
        _GATHER3 = _gather3_build()
    return _GATHER3(k, q, vv, src, dst)


def _scatter_run(ev, s16, dst, zrows):
    global _SCATTER
    if _SCATTER is None:
        _SCATTER = _scatter_build()
    return _SCATTER(ev, s16, dst, zrows)


_SMASK = (np.arange(_D)[:, None] // _DH ==
          np.arange(_H)[None, :]).astype(np.float32)
_BMASK = (np.arange(_D)[None, :] // _DH ==
          np.arange(_H)[:, None]).astype(np.float32)
_BMZ = np.concatenate([_BMASK, np.zeros((8, _D), np.float32)], axis=0)


def _row(x):
    return x.reshape(1, -1)


def kernel(v, e, edge_index, WQ, WK, WV, We, WOv, bOv, WOe, bOe, W1v, b1v,
           W2v, b2v, W1e, b1e, W2e, b2e, g1v, be1v, g1e, be1e, g2v, be2v,
           g2e, be2e):
    src = edge_index[0]
    dst = edge_index[1]
    wqkv = jnp.concatenate([WQ, WK, WV], axis=1)
    we16 = We.astype(_BF16)
    woe16 = WOe.astype(_BF16)
    wov16 = WOv.astype(_BF16)

    q, k, vv = _qkv_call(v, wqkv)

    ksrc, qdst, vsrc = _gather3_run(k, q, vv, src, dst)

    e1, ev, s16, s1, q1 = _edge_a_call(e, ksrc, qdst, vsrc, we16, woe16,
                                       _row(bOe), _SMASK, _BMASK)

    zrows = jnp.zeros((_RPS, 128), _F32)
    wv_pad, oz = _scatter_run(ev, s16.reshape(-1), dst, zrows)
    wv = wv_pad[:_N]
    z16 = jnp.concatenate(
        [oz[:_ZH, :16], oz[_ZP:_ZP + _N - _ZH, :16]], axis=0)

    m1 = s1 / _E
    r1 = lax.rsqrt(q1 / _E - m1 * m1 + 1e-5)
    e2, s2, q2 = _bnffn_call(e1, m1, r1, _row(g1e), _row(be1e), W1e.astype(_BF16),
                             _row(b1e), W2e, _row(b2e), _E, 1000)
    m2 = s2 / _E
    r2 = lax.rsqrt(q2 / _E - m2 * m2 + 1e-5)
    out_e = _bnapply_call(e2, m2, r2, _row(g2e), _row(be2e), _E, 1000)

    v1, sv1, qv1 = _vatt_call(wv, z16, v, wov16, _row(bOv), _BMZ)
    mv1 = sv1 / _N
    rv1 = lax.rsqrt(qv1 / _N - mv1 * mv1 + 1e-5)
    v2, sv2, qv2 = _bnffn_call(v1, mv1, rv1, _row(g1v), _row(be1v), W1v.astype(_BF16),
                               _row(b1v), W2v.astype(_BF16), _row(b2v), _N, 400)
    mv2 = sv2 / _N
    rv2 = lax.rsqrt(qv2 / _N - mv2 * mv2 + 1e-5)
    out_v = _bnapply_call(v2, mv2, rv2, _row(g2v), _row(be2v), _N, 400)

    return (out_v, out_e)


# speedup vs baseline: 3.1206x; 1.2167x over previous
import functools
import numpy as np
import jax
import jax.numpy as jnp
from jax import lax
from jax.experimental import pallas as pl
from jax.experimental.pallas import tpu as pltpu
from jax.experimental.pallas import tpu_sc as plsc

_N = 10000
_E = 160000
_D = 256
_H = 8
_DH = 32
_F32 = jnp.float32
_BF16 = jnp.bfloat16


def _pack_cols(y):
    lo = lax.bitcast_convert_type(y[:, :128].astype(_BF16),
                                  jnp.uint16).astype(jnp.uint32)
    hi = lax.bitcast_convert_type(y[:, 128:].astype(_BF16),
                                  jnp.uint16).astype(jnp.uint32)
    return lax.bitcast_convert_type(lo | (hi << 16), _F32)


def _unpack_cols(x):
    xi = lax.bitcast_convert_type(x, jnp.uint32)
    lo = lax.bitcast_convert_type(xi << 16, _F32)
    hi = lax.bitcast_convert_type(xi & jnp.uint32(0xFFFF0000), _F32)
    return jnp.concatenate([lo, hi], axis=1)


def _qkv_body(v_ref, w_ref, q_ref, k_ref, vv_ref):
    y = jnp.dot(v_ref[...], w_ref[...], preferred_element_type=_F32)
    q_ref[...] = _pack_cols(y[:, :_D])
    k_ref[...] = _pack_cols(y[:, _D:2 * _D])
    vv_ref[...] = _pack_cols(y[:, 2 * _D:])


def _qkv_call(v, wqkv):
    nb = 400
    grid = (_N // nb,)
    return pl.pallas_call(
        _qkv_body,
        grid=grid,
        in_specs=[
            pl.BlockSpec((nb, _D), lambda i: (i, 0)),
            pl.BlockSpec((_D, 3 * _D), lambda i: (0, 0)),
        ],
        out_specs=[
            pl.BlockSpec((nb, 128), lambda i: (i, 0)),
            pl.BlockSpec((nb, 128), lambda i: (i, 0)),
            pl.BlockSpec((nb, 128), lambda i: (i, 0)),
        ],
        out_shape=[jax.ShapeDtypeStruct((_N, 128), _F32)] * 3,
    )(v, wqkv)


def _edge_a_body(e_ref, ks_ref, qd_ref, vs_ref, we_ref, woe_ref, boe_ref,
                 smask_ref, bmask_ref, e1_ref, ev_ref, s16_ref, ssum_ref,
                 ssq_ref):
    eb = e_ref[...]
    pe = jnp.dot(eb.astype(_BF16), we_ref[...], preferred_element_type=_F32)
    kq = _unpack_cols(ks_ref[...]) * _unpack_cols(qd_ref[...])
    score = kq * pe * np.float32(1.0 / np.sqrt(_DH))
    shead = jnp.dot(score, smask_ref[...], preferred_element_type=_F32)
    sexp = jnp.exp(jnp.clip(shead, -5.0, 5.0))
    e1 = eb + jnp.dot(score.astype(_BF16), woe_ref[...],
                      preferred_element_type=_F32) + boe_ref[...]
    e1_ref[...] = e1
    ev_ref[...] = _unpack_cols(vs_ref[...]) * jnp.dot(
        sexp, bmask_ref[...], preferred_element_type=_F32)
    s16_ref[...] = jnp.concatenate(
        [sexp, jnp.zeros((sexp.shape[0], 8), _F32)], axis=1)

    @pl.when(pl.program_id(0) == 0)
    def _():
        ssum_ref[...] = jnp.zeros_like(ssum_ref)
        ssq_ref[...] = jnp.zeros_like(ssq_ref)

    ssum_ref[...] += jnp.sum(e1, axis=0, keepdims=True)
    ssq_ref[...] += jnp.sum(e1 * e1, axis=0, keepdims=True)


def _edge_a_call(e, ksrc, qdst, vsrc, we, woe, boe, smask, bmask):
    eb = 1000
    grid = (_E // eb,)
    big = pl.BlockSpec((eb, _D), lambda i: (i, 0))
    pk = pl.BlockSpec((eb, 128), lambda i: (i, 0))
    return pl.pallas_call(
        _edge_a_body,
        grid=grid,
        in_specs=[
            big, pk, pk, pk,
            pl.BlockSpec((_D, _D), lambda i: (0, 0)),
            pl.BlockSpec((_D, _D), lambda i: (0, 0)),
            pl.BlockSpec((1, _D), lambda i: (0, 0)),
            pl.BlockSpec((_D, _H), lambda i: (0, 0)),
            pl.BlockSpec((_H, _D), lambda i: (0, 0)),
        ],
        out_specs=[
            big, big,
            pl.BlockSpec((eb, 16), lambda i: (i, 0)),
            pl.BlockSpec((1, _D), lambda i: (0, 0)),
            pl.BlockSpec((1, _D), lambda i: (0, 0)),
        ],
        out_shape=[
            jax.ShapeDtypeStruct((_E, _D), _F32),
            jax.ShapeDtypeStruct((_E, _D), _F32),
            jax.ShapeDtypeStruct((_E, 16), _F32),
            jax.ShapeDtypeStruct((1, _D), _F32),
            jax.ShapeDtypeStruct((1, _D), _F32),
        ],
    )(e, ksrc, qdst, vsrc, we, woe, boe, smask, bmask)


def _vatt_body(wv_ref, z_ref, v_ref, wov_ref, bov_ref, bmz_ref, v1_ref,
               ssum_ref, ssq_ref):
    zb = jnp.dot(z_ref[...], bmz_ref[...], preferred_element_type=_F32)
    vatt = wv_ref[...] / (zb + 1e-6)
    v1 = v_ref[...] + jnp.dot(vatt.astype(_BF16), wov_ref[...],
                              preferred_element_type=_F32) + bov_ref[...]
    v1_ref[...] = v1

    @pl.when(pl.program_id(0) == 0)
    def _():
        ssum_ref[...] = jnp.zeros_like(ssum_ref)
        ssq_ref[...] = jnp.zeros_like(ssq_ref)

    ssum_ref[...] += jnp.sum(v1, axis=0, keepdims=True)
    ssq_ref[...] += jnp.sum(v1 * v1, axis=0, keepdims=True)


def _vatt_call(wv, z16, v, wov, bov, bmz):
    nb = 400
    grid = (_N // nb,)
    return pl.pallas_call(
        _vatt_body,
        grid=grid,
        in_specs=[
            pl.BlockSpec((nb, _D), lambda i: (i, 0)),
            pl.BlockSpec((nb, 16), lambda i: (i, 0)),
            pl.BlockSpec((nb, _D), lambda i: (i, 0)),
            pl.BlockSpec((_D, _D), lambda i: (0, 0)),
            pl.BlockSpec((1, _D), lambda i: (0, 0)),
            pl.BlockSpec((16, _D), lambda i: (0, 0)),
        ],
        out_specs=[
            pl.BlockSpec((nb, _D), lambda i: (i, 0)),
            pl.BlockSpec((1, _D), lambda i: (0, 0)),
            pl.BlockSpec((1, _D), lambda i: (0, 0)),
        ],
        out_shape=[
            jax.ShapeDtypeStruct((_N, _D), _F32),
            jax.ShapeDtypeStruct((1, _D), _F32),
            jax.ShapeDtypeStruct((1, _D), _F32),
        ],
    )(wv, z16, v, wov, bov, bmz)


def _bnffn_body(x_ref, m_ref, r_ref, g_ref, b_ref, w1_ref, b1_ref, w2_ref,
                b2_ref, y_ref, ssum_ref, ssq_ref):
    xn = (x_ref[...] - m_ref[...]) * r_ref[...] * g_ref[...] + b_ref[...]
    h = jnp.maximum(
        jnp.dot(xn.astype(_BF16), w1_ref[...],
                preferred_element_type=_F32) + b1_ref[...], 0.0)
    y = xn + jnp.dot(h.astype(_BF16), w2_ref[...],
                     preferred_element_type=_F32) + b2_ref[...]
    y_ref[...] = y

    @pl.when(pl.program_id(0) == 0)
    def _():
        ssum_ref[...] = jnp.zeros_like(ssum_ref)
        ssq_ref[...] = jnp.zeros_like(ssq_ref)

    ssum_ref[...] += jnp.sum(y, axis=0, keepdims=True)
    ssq_ref[...] += jnp.sum(y * y, axis=0, keepdims=True)


def _bnffn_call(x, m, r, g, b, w1, b1, w2, b2, rows, rb):
    grid = (rows // rb,)
    vec = pl.BlockSpec((1, _D), lambda i: (0, 0))
    return pl.pallas_call(
        _bnffn_body,
        grid=grid,
        in_specs=[
            pl.BlockSpec((rb, _D), lambda i: (i, 0)),
            vec, vec, vec, vec,
            pl.BlockSpec((_D, 2 * _D), lambda i: (0, 0)),
            pl.BlockSpec((1, 2 * _D), lambda i: (0, 0)),
            pl.BlockSpec((2 * _D, _D), lambda i: (0, 0)),
            vec,
        ],
        out_specs=[
            pl.BlockSpec((rb, _D), lambda i: (i, 0)),
            pl.BlockSpec((1, _D), lambda i: (0, 0)),
            pl.BlockSpec((1, _D), lambda i: (0, 0)),
        ],
        out_shape=[
            jax.ShapeDtypeStruct((rows, _D), _F32),
            jax.ShapeDtypeStruct((1, _D), _F32),
            jax.ShapeDtypeStruct((1, _D), _F32),
        ],
    )(x, m, r, g, b, w1, b1, w2, b2)


def _bnapply_body(x_ref, m_ref, r_ref, g_ref, b_ref, y_ref):
    y_ref[...] = (x_ref[...] - m_ref[...]) * r_ref[...] * g_ref[...] \
        + b_ref[...]


def _bnapply_call(x, m, r, g, b, rows, rb):
    grid = (rows // rb,)
    vec = pl.BlockSpec((1, _D), lambda i: (0, 0))
    return pl.pallas_call(
        _bnapply_body,
        grid=grid,
        in_specs=[pl.BlockSpec((rb, _D), lambda i: (i, 0)), vec, vec, vec,
                  vec],
        out_specs=pl.BlockSpec((rb, _D), lambda i: (i, 0)),
        out_shape=jax.ShapeDtypeStruct((rows, _D), _F32),
    )(x, m, r, g, b)


_NC = 2
_NS = 16
_NW = _NC * _NS
_GC = 128
_DP = 128
_NCH = _E // _GC
_GJ = _NCH // _NW
_GT = _NCH - _GJ * _NW
_SC = 128
_SJ = _NCH // _NS
_ST = _NCH - _SJ * _NS
_NP = 10240
_RPS = _NP // _NS
_ZH = _NP // _NC
_ZP = 5248
_ZRS = _ZP // _NS


def _gather3_build():
    mesh = plsc.VectorSubcoreMesh(core_axis_name="c", subcore_axis_name="s", num_cores=_NC, num_subcores=_NS)

    @functools.partial(
        pl.kernel,
        out_type=(
            jax.ShapeDtypeStruct((_E, _DP), _F32),
            jax.ShapeDtypeStruct((_E, _DP), _F32),
            jax.ShapeDtypeStruct((_E, _DP), _F32),
        ),
        mesh=mesh,
        scratch_types=[
            pltpu.VMEM((_GC,), jnp.int32),
            pltpu.VMEM((_GC,), jnp.int32),
            pltpu.VMEM((_GC, _DP), _F32),
            pltpu.VMEM((_GC, _DP), _F32),
            pltpu.VMEM((_GC, _DP), _F32),
            pltpu.SemaphoreType.DMA,
        ],
    )
    def gather3(ktab, qtab, vtab, src, dst, ok, oq, ov, src_v, dst_v, bk, bq,
                bv, sem):
        wid = lax.axis_index("s") * _NC + lax.axis_index("c")

        def do_chunk(off):
            pltpu.sync_copy(src.at[pl.ds(off, _GC)], src_v)
            pltpu.sync_copy(dst.at[pl.ds(off, _GC)], dst_v)
            ck = pltpu.async_copy(ktab.at[src_v], bk, sem)
            cq = pltpu.async_copy(qtab.at[dst_v], bq, sem)
            cv = pltpu.async_copy(vtab.at[src_v], bv, sem)
            ck.wait()
            cq.wait()
            cv.wait()
            pltpu.sync_copy(bk, ok.at[pl.ds(off, _GC)])
            pltpu.sync_copy(bq, oq.at[pl.ds(off, _GC)])
            pltpu.sync_copy(bv, ov.at[pl.ds(off, _GC)])

        def body(j, carry):
            do_chunk((wid + _NW * j) * _GC)
            return carry

        lax.fori_loop(0, _GJ, body, 0)

        @pl.when(wid < _GT)
        def _():
            do_chunk((_GJ * _NW + wid) * _GC)

    return gather3


def _scatter_build():
    mesh = plsc.VectorSubcoreMesh(core_axis_name="c", subcore_axis_name="s", num_cores=_NC, num_subcores=_NS)

    @functools.partial(
        pl.kernel,
        out_type=(
            jax.ShapeDtypeStruct((_NP, _D), _F32),
            jax.ShapeDtypeStruct((_NC * _ZP, 128), _F32),
        ),
        mesh=mesh,
        scratch_types=[
            pltpu.VMEM((_SC,), jnp.int32),
            pltpu.VMEM((_SC,), jnp.int32),
            pltpu.VMEM((_SC, 128), _F32),
            pltpu.VMEM((_SC, 128), _F32),
            pltpu.VMEM((_SC * 16,), _F32),
            pltpu.VMEM_SHARED((_NP, 128), _F32),
        ],
    )
    def scatter(ev, s16, dst, zrows, owv, oz, dst_v, dstc_v, evb, zb, sb,
                acc):
        cid = lax.axis_index("c")
        sid = lax.axis_index("s")
        col0 = cid * 128
        node0 = cid * _ZH

        pltpu.sync_copy(zrows, acc.at[pl.ds(sid * _RPS, _RPS)])
        plsc.subcore_barrier()

        def chunk_ev(off):
            pltpu.sync_copy(dst.at[pl.ds(off, _SC)], dst_v)
            pltpu.sync_copy(ev.at[pl.ds(off, _SC), pl.ds(col0, 128)], evb)
            pltpu.sync_copy(evb, acc.at[dst_v], add=True)

        def body_ev(j, carry):
            chunk_ev((sid + _NS * j) * _SC)
            return carry

        lax.fori_loop(0, _SJ, body_ev, 0)

        @pl.when(sid < _ST)
        def _():
            chunk_ev((_SJ * _NS + sid) * _SC)
        plsc.subcore_barrier()
        r0 = sid * _RPS
        pltpu.sync_copy(acc.at[pl.ds(r0, _RPS)],
                        owv.at[pl.ds(r0, _RPS), pl.ds(col0, 128)])
        plsc.subcore_barrier()

        pltpu.sync_copy(zrows.at[pl.ds(0, _ZRS)],
                        acc.at[pl.ds(sid * _ZRS, _ZRS)])
        plsc.subcore_barrier()

        pltpu.sync_copy(zrows.at[pl.ds(0, _SC)], zb)

        def chunk_z(off):
            pltpu.sync_copy(dst.at[pl.ds(off, _SC)], dst_v)
            pltpu.sync_copy(s16.at[pl.ds(off * 16, _SC * 16)], sb)
            for t in range(_SC):
                zb[t, pl.ds(0, 16)] = sb[pl.ds(t * 16, 16)]
            for t in range(_SC // 16):
                iv = dst_v[pl.ds(t * 16, 16)]
                rel = iv - node0
                good = (rel >= 0) & (rel < _ZH)
                dstc_v[pl.ds(t * 16, 16)] = jnp.where(good, rel, _ZH)
            pltpu.sync_copy(zb, acc.at[dstc_v], add=True)

        def body_z(j, carry):
            chunk_z((sid + _NS * j) * _SC)
            return carry

        lax.fori_loop(0, _SJ, body_z, 0)

        @pl.when(sid < _ST)
        def _():
            chunk_z((_SJ * _NS + sid) * _SC)
        plsc.subcore_barrier()
        rz = sid * _ZRS
        pltpu.sync_copy(acc.at[pl.ds(rz, _ZRS)],
                        oz.at[pl.ds(cid * _ZP + rz, _ZRS)])

    return scatter


_GATHER3 = None
_SCATTER = None


def _gather3_run(k, q, vv, src, dst):
    global _GATHER3
    if _GATHER3 is None:
        _GATHER3 = _gather3_build()
    return _GATHER3(k, q, vv, src, dst)


def _scatter_run(ev, s16, dst, zrows):
    global _SCATTER
    if _SCATTER is None:
        _SCATTER = _scatter_build()
    return _SCATTER(ev, s16, dst, zrows)


_SMASK = (np.arange(_D)[:, None] // _DH ==
          np.arange(_H)[None, :]).astype(np.float32)
_BMASK = (np.arange(_D)[None, :] // _DH ==
          np.arange(_H)[:, None]).astype(np.float32)
_BMZ = np.concatenate([_BMASK, np.zeros((8, _D), np.float32)], axis=0)


def _row(x):
    return x.reshape(1, -1)


def kernel(v, e, edge_index, WQ, WK, WV, We, WOv, bOv, WOe, bOe, W1v, b1v,
           W2v, b2v, W1e, b1e, W2e, b2e, g1v, be1v, g1e, be1e, g2v, be2v,
           g2e, be2e):
    src = edge_index[0]
    dst = edge_index[1]
    wqkv = jnp.concatenate([WQ, WK, WV], axis=1)
    we16 = We.astype(_BF16)
    woe16 = WOe.astype(_BF16)
    wov16 = WOv.astype(_BF16)

    q, k, vv = _qkv_call(v, wqkv)

    ksrc, qdst, vsrc = _gather3_run(k, q, vv, src, dst)

    e1, ev, s16, s1, q1 = _edge_a_call(e, ksrc, qdst, vsrc, we16, woe16,
                                       _row(bOe), _SMASK, _BMASK)

    zrows = jnp.zeros((_RPS, 128), _F32)
    wv_pad, oz = _scatter_run(ev, s16.reshape(-1), dst, zrows)
    wv = wv_pad[:_N]
    z16 = jnp.concatenate(
        [oz[:_ZH, :16], oz[_ZP:_ZP + _N - _ZH, :16]], axis=0)

    m1 = s1 / _E
    r1 = lax.rsqrt(q1 / _E - m1 * m1 + 1e-5)
    e2, s2, q2 = _bnffn_call(e1, m1, r1, _row(g1e), _row(be1e), W1e.astype(_BF16),
                             _row(b1e), W2e, _row(b2e), _E, 1000)
    m2 = s2 / _E
    r2 = lax.rsqrt(q2 / _E - m2 * m2 + 1e-5)
    out_e = _bnapply_call(e2, m2, r2, _row(g2e), _row(be2e), _E, 1000)

    v1, sv1, qv1 = _vatt_call(wv, z16, v, wov16, _row(bOv), _BMZ)
    mv1 = sv1 / _N
    rv1 = lax.rsqrt(qv1 / _N - mv1 * mv1 + 1e-5)
    v2, sv2, qv2 = _bnffn_call(v1, mv1, rv1, _row(g1v), _row(be1v), W1v.astype(_BF16),
                               _row(b1v), W2v.astype(_BF16), _row(b2v), _N, 400)
    mv2 = sv2 / _N
    rv2 = lax.rsqrt(qv2 / _N - mv2 * mv2 + 1e-5)
    out_v = _bnapply_call(v2, mv2, rv2, _row(g2v), _row(be2v), _N, 400)

    return (out_v, out_e)


# speedup vs baseline: 3.1930x; 1.0232x over previous
import functools
import numpy as np
import jax
import jax.numpy as jnp
from jax import lax
from jax.experimental import pallas as pl
from jax.experimental.pallas import tpu as pltpu
from jax.experimental.pallas import tpu_sc as plsc

_N = 10000
_E = 160000
_D = 256
_H = 8
_DH = 32
_F32 = jnp.float32
_BF16 = jnp.bfloat16


def _pack_cols(y):
    lo = lax.bitcast_convert_type(y[:, :128].astype(_BF16),
                                  jnp.uint16).astype(jnp.uint32)
    hi = lax.bitcast_convert_type(y[:, 128:].astype(_BF16),
                                  jnp.uint16).astype(jnp.uint32)
    return lax.bitcast_convert_type(lo | (hi << 16), _F32)


def _unpack_cols(x):
    xi = lax.bitcast_convert_type(x, jnp.uint32)
    lo = lax.bitcast_convert_type(xi << 16, _F32)
    hi = lax.bitcast_convert_type(xi & jnp.uint32(0xFFFF0000), _F32)
    return jnp.concatenate([lo, hi], axis=1)


def _qkv_body(v_ref, w_ref, q_ref, k_ref, vv_ref):
    y = jnp.dot(v_ref[...], w_ref[...], preferred_element_type=_F32)
    q_ref[...] = _pack_cols(y[:, :_D])
    k_ref[...] = _pack_cols(y[:, _D:2 * _D])
    vv_ref[...] = _pack_cols(y[:, 2 * _D:])


def _qkv_call(v, wqkv):
    nb = 400
    grid = (_N // nb,)
    return pl.pallas_call(
        _qkv_body,
        grid=grid,
        in_specs=[
            pl.BlockSpec((nb, _D), lambda i: (i, 0)),
            pl.BlockSpec((_D, 3 * _D), lambda i: (0, 0)),
        ],
        out_specs=[
            pl.BlockSpec((nb, 128), lambda i: (i, 0)),
            pl.BlockSpec((nb, 128), lambda i: (i, 0)),
            pl.BlockSpec((nb, 128), lambda i: (i, 0)),
        ],
        out_shape=[jax.ShapeDtypeStruct((_N, 128), _F32)] * 3,
    )(v, wqkv)


def _edge_a_body(e_ref, ks_ref, qd_ref, vs_ref, we_ref, woe_ref, boe_ref,
                 smask_ref, bmask_ref, e1_ref, ev_ref, s16_ref, ssum_ref,
                 ssq_ref):
    eb = e_ref[...]
    pe = jnp.dot(eb.astype(_BF16), we_ref[...], preferred_element_type=_F32)
    kq = _unpack_cols(ks_ref[...]) * _unpack_cols(qd_ref[...])
    score = kq * pe * np.float32(1.0 / np.sqrt(_DH))
    shead = jnp.dot(score, smask_ref[...], preferred_element_type=_F32)
    sexp = jnp.exp(jnp.clip(shead, -5.0, 5.0))
    e1 = eb + jnp.dot(score.astype(_BF16), woe_ref[...],
                      preferred_element_type=_F32) + boe_ref[...]
    e1_ref[...] = e1.astype(e1_ref.dtype)
    ev_ref[...] = _unpack_cols(vs_ref[...]) * jnp.dot(
        sexp, bmask_ref[...], preferred_element_type=_F32)
    s16_ref[...] = jnp.concatenate(
        [sexp, jnp.zeros((sexp.shape[0], 8), _F32)], axis=1)

    @pl.when(pl.program_id(0) == 0)
    def _():
        ssum_ref[...] = jnp.zeros_like(ssum_ref)
        ssq_ref[...] = jnp.zeros_like(ssq_ref)

    ssum_ref[...] += jnp.sum(e1, axis=0, keepdims=True)
    ssq_ref[...] += jnp.sum(e1 * e1, axis=0, keepdims=True)


def _edge_a_call(e, ksrc, qdst, vsrc, we, woe, boe, smask, bmask):
    eb = 1000
    grid = (_E // eb,)
    big = pl.BlockSpec((eb, _D), lambda i: (i, 0))
    pk = pl.BlockSpec((eb, 128), lambda i: (i, 0))
    return pl.pallas_call(
        _edge_a_body,
        grid=grid,
        in_specs=[
            big, pk, pk, pk,
            pl.BlockSpec((_D, _D), lambda i: (0, 0)),
            pl.BlockSpec((_D, _D), lambda i: (0, 0)),
            pl.BlockSpec((1, _D), lambda i: (0, 0)),
            pl.BlockSpec((_D, _H), lambda i: (0, 0)),
            pl.BlockSpec((_H, _D), lambda i: (0, 0)),
        ],
        out_specs=[
            big, big,
            pl.BlockSpec((eb, 16), lambda i: (i, 0)),
            pl.BlockSpec((1, _D), lambda i: (0, 0)),
            pl.BlockSpec((1, _D), lambda i: (0, 0)),
        ],
        out_shape=[
            jax.ShapeDtypeStruct((_E, _D), _BF16),
            jax.ShapeDtypeStruct((_E, _D), _F32),
            jax.ShapeDtypeStruct((_E, 16), _F32),
            jax.ShapeDtypeStruct((1, _D), _F32),
            jax.ShapeDtypeStruct((1, _D), _F32),
        ],
    )(e, ksrc, qdst, vsrc, we, woe, boe, smask, bmask)


def _vatt_body(wv_ref, z_ref, v_ref, wov_ref, bov_ref, bmz_ref, v1_ref,
               ssum_ref, ssq_ref):
    zb = jnp.dot(z_ref[...], bmz_ref[...], preferred_element_type=_F32)
    vatt = wv_ref[...] / (zb + 1e-6)
    v1 = v_ref[...] + jnp.dot(vatt.astype(_BF16), wov_ref[...],
                              preferred_element_type=_F32) + bov_ref[...]
    v1_ref[...] = v1

    @pl.when(pl.program_id(0) == 0)
    def _():
        ssum_ref[...] = jnp.zeros_like(ssum_ref)
        ssq_ref[...] = jnp.zeros_like(ssq_ref)

    ssum_ref[...] += jnp.sum(v1, axis=0, keepdims=True)
    ssq_ref[...] += jnp.sum(v1 * v1, axis=0, keepdims=True)


def _vatt_call(wv, z16, v, wov, bov, bmz):
    nb = 400
    grid = (_N // nb,)
    return pl.pallas_call(
        _vatt_body,
        grid=grid,
        in_specs=[
            pl.BlockSpec((nb, _D), lambda i: (i, 0)),
            pl.BlockSpec((nb, 16), lambda i: (i, 0)),
            pl.BlockSpec((nb, _D), lambda i: (i, 0)),
            pl.BlockSpec((_D, _D), lambda i: (0, 0)),
            pl.BlockSpec((1, _D), lambda i: (0, 0)),
            pl.BlockSpec((16, _D), lambda i: (0, 0)),
        ],
        out_specs=[
            pl.BlockSpec((nb, _D), lambda i: (i, 0)),
            pl.BlockSpec((1, _D), lambda i: (0, 0)),
            pl.BlockSpec((1, _D), lambda i: (0, 0)),
        ],
        out_shape=[
            jax.ShapeDtypeStruct((_N, _D), _F32),
            jax.ShapeDtypeStruct((1, _D), _F32),
            jax.ShapeDtypeStruct((1, _D), _F32),
        ],
    )(wv, z16, v, wov, bov, bmz)


def _bnffn_body(x_ref, m_ref, r_ref, g_ref, b_ref, w1_ref, b1_ref, w2_ref,
                b2_ref, y_ref, ssum_ref, ssq_ref):
    xn = (x_ref[...].astype(_F32) - m_ref[...]) * r_ref[...] * g_ref[...] \
        + b_ref[...]
    h = jnp.maximum(
        jnp.dot(xn.astype(_BF16), w1_ref[...],
                preferred_element_type=_F32) + b1_ref[...], 0.0)
    y = xn + jnp.dot(h.astype(_BF16), w2_ref[...],
                     preferred_element_type=_F32) + b2_ref[...]
    y_ref[...] = y.astype(y_ref.dtype)

    @pl.when(pl.program_id(0) == 0)
    def _():
        ssum_ref[...] = jnp.zeros_like(ssum_ref)
        ssq_ref[...] = jnp.zeros_like(ssq_ref)

    ssum_ref[...] += jnp.sum(y, axis=0, keepdims=True)
    ssq_ref[...] += jnp.sum(y * y, axis=0, keepdims=True)


def _bnffn_call(x, m, r, g, b, w1, b1, w2, b2, rows, rb, odt=_F32):
    grid = (rows // rb,)
    vec = pl.BlockSpec((1, _D), lambda i: (0, 0))
    return pl.pallas_call(
        _bnffn_body,
        grid=grid,
        in_specs=[
            pl.BlockSpec((rb, _D), lambda i: (i, 0)),
            vec, vec, vec, vec,
            pl.BlockSpec((_D, 2 * _D), lambda i: (0, 0)),
            pl.BlockSpec((1, 2 * _D), lambda i: (0, 0)),
            pl.BlockSpec((2 * _D, _D), lambda i: (0, 0)),
            vec,
        ],
        out_specs=[
            pl.BlockSpec((rb, _D), lambda i: (i, 0)),
            pl.BlockSpec((1, _D), lambda i: (0, 0)),
            pl.BlockSpec((1, _D), lambda i: (0, 0)),
        ],
        out_shape=[
            jax.ShapeDtypeStruct((rows, _D), odt),
            jax.ShapeDtypeStruct((1, _D), _F32),
            jax.ShapeDtypeStruct((1, _D), _F32),
        ],
    )(x, m, r, g, b, w1, b1, w2, b2)


def _bnapply_body(x_ref, m_ref, r_ref, g_ref, b_ref, y_ref):
    y_ref[...] = (x_ref[...].astype(_F32) - m_ref[...]) * r_ref[...] \
        * g_ref[...] + b_ref[...]


def _bnapply_call(x, m, r, g, b, rows, rb):
    grid = (rows // rb,)
    vec = pl.BlockSpec((1, _D), lambda i: (0, 0))
    return pl.pallas_call(
        _bnapply_body,
        grid=grid,
        in_specs=[pl.BlockSpec((rb, _D), lambda i: (i, 0)), vec, vec, vec,
                  vec],
        out_specs=pl.BlockSpec((rb, _D), lambda i: (i, 0)),
        out_shape=jax.ShapeDtypeStruct((rows, _D), _F32),
    )(x, m, r, g, b)


_NC = 2
_NS = 16
_NW = _NC * _NS
_GC = 128
_DP = 128
_NCH = _E // _GC
_GJ = _NCH // _NW
_GT = _NCH - _GJ * _NW
_SC = 128
_SJ = _NCH // _NS
_ST = _NCH - _SJ * _NS
_NP = 10240
_RPS = _NP // _NS
_ZH = _NP // _NC
_ZP = 5248
_ZRS = _ZP // _NS


def _gather3_build():
    mesh = plsc.VectorSubcoreMesh(core_axis_name="c", subcore_axis_name="s", num_cores=_NC, num_subcores=_NS)

    @functools.partial(
        pl.kernel,
        out_type=(
            jax.ShapeDtypeStruct((_E, _DP), _F32),
            jax.ShapeDtypeStruct((_E, _DP), _F32),
            jax.ShapeDtypeStruct((_E, _DP), _F32),
        ),
        mesh=mesh,
        scratch_types=[
            pltpu.VMEM((_GC,), jnp.int32),
            pltpu.VMEM((_GC,), jnp.int32),
            pltpu.VMEM((_GC, _DP), _F32),
            pltpu.VMEM((_GC, _DP), _F32),
            pltpu.VMEM((_GC, _DP), _F32),
            pltpu.SemaphoreType.DMA,
        ],
    )
    def gather3(ktab, qtab, vtab, src, dst, ok, oq, ov, src_v, dst_v, bk, bq,
                bv, sem):
        wid = lax.axis_index("s") * _NC + lax.axis_index("c")

        def do_chunk(off):
            pltpu.sync_copy(src.at[pl.ds(off, _GC)], src_v)
            pltpu.sync_copy(dst.at[pl.ds(off, _GC)], dst_v)
            ck = pltpu.async_copy(ktab.at[src_v], bk, sem)
            cq = pltpu.async_copy(qtab.at[dst_v], bq, sem)
            cv = pltpu.async_copy(vtab.at[src_v], bv, sem)
            ck.wait()
            cq.wait()
            cv.wait()
            pltpu.sync_copy(bk, ok.at[pl.ds(off, _GC)])
            pltpu.sync_copy(bq, oq.at[pl.ds(off, _GC)])
            pltpu.sync_copy(bv, ov.at[pl.ds(off, _GC)])

        def body(j, carry):
            do_chunk((wid + _NW * j) * _GC)
            return carry

        lax.fori_loop(0, _GJ, body, 0)

        @pl.when(wid < _GT)
        def _():
            do_chunk((_GJ * _NW + wid) * _GC)

    return gather3


def _scatter_build():
    mesh = plsc.VectorSubcoreMesh(core_axis_name="c", subcore_axis_name="s", num_cores=_NC, num_subcores=_NS)

    @functools.partial(
        pl.kernel,
        out_type=(
            jax.ShapeDtypeStruct((_NP, _D), _F32),
            jax.ShapeDtypeStruct((_NC * _ZP, 128), _F32),
        ),
        mesh=mesh,
        scratch_types=[
            pltpu.VMEM((_SC,), jnp.int32),
            pltpu.VMEM((_SC,), jnp.int32),
            pltpu.VMEM((_SC, 128), _F32),
            pltpu.VMEM((_SC, 128), _F32),
            pltpu.VMEM((_SC * 16,), _F32),
            pltpu.VMEM_SHARED((_NP, 128), _F32),
        ],
    )
    def scatter(ev, s16, dst, zrows, owv, oz, dst_v, dstc_v, evb, zb, sb,
                acc):
        cid = lax.axis_index("c")
        sid = lax.axis_index("s")
        col0 = cid * 128
        node0 = cid * _ZH

        pltpu.sync_copy(zrows, acc.at[pl.ds(sid * _RPS, _RPS)])
        plsc.subcore_barrier()

        def chunk_ev(off):
            pltpu.sync_copy(dst.at[pl.ds(off, _SC)], dst_v)
            pltpu.sync_copy(ev.at[pl.ds(off, _SC), pl.ds(col0, 128)], evb)
            pltpu.sync_copy(evb, acc.at[dst_v], add=True)

        def body_ev(j, carry):
            chunk_ev((sid + _NS * j) * _SC)
            return carry

        lax.fori_loop(0, _SJ, body_ev, 0)

        @pl.when(sid < _ST)
        def _():
            chunk_ev((_SJ * _NS + sid) * _SC)
        plsc.subcore_barrier()
        r0 = sid * _RPS
        pltpu.sync_copy(acc.at[pl.ds(r0, _RPS)],
                        owv.at[pl.ds(r0, _RPS), pl.ds(col0, 128)])
        plsc.subcore_barrier()

        pltpu.sync_copy(zrows.at[pl.ds(0, _ZRS)],
                        acc.at[pl.ds(sid * _ZRS, _ZRS)])
        plsc.subcore_barrier()

        pltpu.sync_copy(zrows.at[pl.ds(0, _SC)], zb)

        def chunk_z(off):
            pltpu.sync_copy(dst.at[pl.ds(off, _SC)], dst_v)
            pltpu.sync_copy(s16.at[pl.ds(off * 16, _SC * 16)], sb)
            for t in range(_SC):
                zb[t, pl.ds(0, 16)] = sb[pl.ds(t * 16, 16)]
            for t in range(_SC // 16):
                iv = dst_v[pl.ds(t * 16, 16)]
                rel = iv - node0
                good = (rel >= 0) & (rel < _ZH)
                dstc_v[pl.ds(t * 16, 16)] = jnp.where(good, rel, _ZH)
            pltpu.sync_copy(zb, acc.at[dstc_v], add=True)

        def body_z(j, carry):
            chunk_z((sid + _NS * j) * _SC)
            return carry

        lax.fori_loop(0, _SJ, body_z, 0)

        @pl.when(sid < _ST)
        def _():
            chunk_z((_SJ * _NS + sid) * _SC)
        plsc.subcore_barrier()
        rz = sid * _ZRS
        pltpu.sync_copy(acc.at[pl.ds(rz, _ZRS)],
                        oz.at[pl.ds(cid * _ZP + rz, _ZRS)])

    return scatter


_GATHER3 = None
_SCATTER = None


def _gather3_run(k, q, vv, src, dst):
    global _GATHER3
    if _GATHER3 is None:
        _GATHER3 = _gather3_build()
    return _GATHER3(k, q, vv, src, dst)


def _scatter_run(ev, s16, dst, zrows):
    global _SCATTER
    if _SCATTER is None:
        _SCATTER = _scatter_build()
    return _SCATTER(ev, s16, dst, zrows)


_SMASK = (np.arange(_D)[:, None] // _DH ==
          np.arange(_H)[None, :]).astype(np.float32)
_BMASK = (np.arange(_D)[None, :] // _DH ==
          np.arange(_H)[:, None]).astype(np.float32)
_BMZ = np.concatenate([_BMASK, np.zeros((8, _D), np.float32)], axis=0)


def _row(x):
    return x.reshape(1, -1)


def kernel(v, e, edge_index, WQ, WK, WV, We, WOv, bOv, WOe, bOe, W1v, b1v,
           W2v, b2v, W1e, b1e, W2e, b2e, g1v, be1v, g1e, be1e, g2v, be2v,
           g2e, be2e):
    src = edge_index[0]
    dst = edge_index[1]
    wqkv = jnp.concatenate([WQ, WK, WV], axis=1)
    we16 = We.astype(_BF16)
    woe16 = WOe.astype(_BF16)
    wov16 = WOv.astype(_BF16)

    q, k, vv = _qkv_call(v, wqkv)

    ksrc, qdst, vsrc = _gather3_run(k, q, vv, src, dst)

    e1, ev, s16, s1, q1 = _edge_a_call(e, ksrc, qdst, vsrc, we16, woe16,
                                       _row(bOe), _SMASK, _BMASK)

    zrows = jnp.zeros((_RPS, 128), _F32)
    wv_pad, oz = _scatter_run(ev, s16.reshape(-1), dst, zrows)
    wv = wv_pad[:_N]
    z16 = jnp.concatenate(
        [oz[:_ZH, :16], oz[_ZP:_ZP + _N - _ZH, :16]], axis=0)

    m1 = s1 / _E
    r1 = lax.rsqrt(q1 / _E - m1 * m1 + 1e-5)
    e2, s2, q2 = _bnffn_call(e1, m1, r1, _row(g1e), _row(be1e), W1e.astype(_BF16),
                             _row(b1e), W2e, _row(b2e), _E, 1000)
    m2 = s2 / _E
    r2 = lax.rsqrt(q2 / _E - m2 * m2 + 1e-5)
    out_e = _bnapply_call(e2, m2, r2, _row(g2e), _row(be2e), _E, 1000)

    v1, sv1, qv1 = _vatt_call(wv, z16, v, wov16, _row(bOv), _BMZ)
    mv1 = sv1 / _N
    rv1 = lax.rsqrt(qv1 / _N - mv1 * mv1 + 1e-5)
    v2, sv2, qv2 = _bnffn_call(v1, mv1, rv1, _row(g1v), _row(be1v), W1v.astype(_BF16),
                               _row(b1v), W2v.astype(_BF16), _row(b2v), _N, 400)
    mv2 = sv2 / _N
    rv2 = lax.rsqrt(qv2 / _N - mv2 * mv2 + 1e-5)
    out_v = _bnapply_call(v2, mv2, rv2, _row(g2v), _row(be2v), _N, 400)

    return (out_v, out_e)


# speedup vs baseline: 3.2876x; 1.0296x over previous
import functools
import numpy as np
import jax
import jax.numpy as jnp
from jax import lax
from jax.experimental import pallas as pl
from jax.experimental.pallas import tpu as pltpu
from jax.experimental.pallas import tpu_sc as plsc

_N = 10000
_E = 160000
_D = 256
_H = 8
_DH = 32
_F32 = jnp.float32
_BF16 = jnp.bfloat16


def _pack_cols(y):
    lo = lax.bitcast_convert_type(y[:, :128].astype(_BF16),
                                  jnp.uint16).astype(jnp.uint32)
    hi = lax.bitcast_convert_type(y[:, 128:].astype(_BF16),
                                  jnp.uint16).astype(jnp.uint32)
    return lax.bitcast_convert_type(lo | (hi << 16), _F32)


def _unpack_cols(x):
    xi = lax.bitcast_convert_type(x, jnp.uint32)
    lo = lax.bitcast_convert_type(xi << 16, _F32)
    hi = lax.bitcast_convert_type(xi & jnp.uint32(0xFFFF0000), _F32)
    return jnp.concatenate([lo, hi], axis=1)


def _qkv_body(v_ref, w_ref, q_ref, k_ref, vv_ref):
    y = jnp.dot(v_ref[...], w_ref[...], preferred_element_type=_F32)
    q_ref[...] = _pack_cols(y[:, :_D])
    k_ref[...] = _pack_cols(y[:, _D:2 * _D])
    vv_ref[...] = _pack_cols(y[:, 2 * _D:])


def _qkv_call(v, wqkv):
    nb = 400
    grid = (_N // nb,)
    return pl.pallas_call(
        _qkv_body,
        grid=grid,
        in_specs=[
            pl.BlockSpec((nb, _D), lambda i: (i, 0)),
            pl.BlockSpec((_D, 3 * _D), lambda i: (0, 0)),
        ],
        out_specs=[
            pl.BlockSpec((nb, 128), lambda i: (i, 0)),
            pl.BlockSpec((nb, 128), lambda i: (i, 0)),
            pl.BlockSpec((nb, 128), lambda i: (i, 0)),
        ],
        out_shape=[jax.ShapeDtypeStruct((_N, 128), _F32)] * 3,
    )(v, wqkv)


def _edge_a_body(e_ref, ks_ref, qd_ref, vs_ref, we_ref, woe_ref, boe_ref,
                 smask_ref, bmask_ref, e1_ref, ev_ref, s16_ref, ssum_ref,
                 ssq_ref):
    eb = e_ref[...]
    pe = jnp.dot(eb.astype(_BF16), we_ref[...], preferred_element_type=_F32)
    kq = _unpack_cols(ks_ref[...]) * _unpack_cols(qd_ref[...])
    score = kq * pe * np.float32(1.0 / np.sqrt(_DH))
    shead = jnp.dot(score, smask_ref[...], preferred_element_type=_F32)
    sexp = jnp.exp(jnp.clip(shead, -5.0, 5.0))
    e1 = eb + jnp.dot(score.astype(_BF16), woe_ref[...],
                      preferred_element_type=_F32) + boe_ref[...]
    e1_ref[...] = e1.astype(e1_ref.dtype)
    ev_ref[...] = _unpack_cols(vs_ref[...]) * jnp.dot(
        sexp, bmask_ref[...], preferred_element_type=_F32)
    s16_ref[...] = jnp.concatenate(
        [sexp, jnp.zeros((sexp.shape[0], 8), _F32)], axis=1)

    @pl.when(pl.program_id(0) == 0)
    def _():
        ssum_ref[...] = jnp.zeros_like(ssum_ref)
        ssq_ref[...] = jnp.zeros_like(ssq_ref)

    ssum_ref[...] += jnp.sum(e1, axis=0, keepdims=True)
    ssq_ref[...] += jnp.sum(e1 * e1, axis=0, keepdims=True)


def _edge_a_call(e, ksrc, qdst, vsrc, we, woe, boe, smask, bmask):
    eb = 1000
    grid = (_E // eb,)
    big = pl.BlockSpec((eb, _D), lambda i: (i, 0))
    pk = pl.BlockSpec((eb, 128), lambda i: (i, 0))
    return pl.pallas_call(
        _edge_a_body,
        grid=grid,
        in_specs=[
            big, pk, pk, pk,
            pl.BlockSpec((_D, _D), lambda i: (0, 0)),
            pl.BlockSpec((_D, _D), lambda i: (0, 0)),
            pl.BlockSpec((1, _D), lambda i: (0, 0)),
            pl.BlockSpec((_D, _H), lambda i: (0, 0)),
            pl.BlockSpec((_H, _D), lambda i: (0, 0)),
        ],
        out_specs=[
            big, big,
            pl.BlockSpec((eb, 16), lambda i: (i, 0)),
            pl.BlockSpec((1, _D), lambda i: (0, 0)),
            pl.BlockSpec((1, _D), lambda i: (0, 0)),
        ],
        out_shape=[
            jax.ShapeDtypeStruct((_E, _D), _BF16),
            jax.ShapeDtypeStruct((_E, _D), _F32),
            jax.ShapeDtypeStruct((_E, 16), _F32),
            jax.ShapeDtypeStruct((1, _D), _F32),
            jax.ShapeDtypeStruct((1, _D), _F32),
        ],
    )(e, ksrc, qdst, vsrc, we, woe, boe, smask, bmask)


def _vatt_body(wv_ref, z_ref, v_ref, wov_ref, bov_ref, bmz_ref, v1_ref,
               ssum_ref, ssq_ref):
    zb = jnp.dot(z_ref[...], bmz_ref[...], preferred_element_type=_F32)
    vatt = wv_ref[...] / (zb + 1e-6)
    v1 = v_ref[...] + jnp.dot(vatt.astype(_BF16), wov_ref[...],
                              preferred_element_type=_F32) + bov_ref[...]
    v1_ref[...] = v1

    @pl.when(pl.program_id(0) == 0)
    def _():
        ssum_ref[...] = jnp.zeros_like(ssum_ref)
        ssq_ref[...] = jnp.zeros_like(ssq_ref)

    ssum_ref[...] += jnp.sum(v1, axis=0, keepdims=True)
    ssq_ref[...] += jnp.sum(v1 * v1, axis=0, keepdims=True)


def _vatt_call(wv, z16, v, wov, bov, bmz):
    nb = 400
    grid = (_N // nb,)
    return pl.pallas_call(
        _vatt_body,
        grid=grid,
        in_specs=[
            pl.BlockSpec((nb, _D), lambda i: (i, 0)),
            pl.BlockSpec((nb, 16), lambda i: (i, 0)),
            pl.BlockSpec((nb, _D), lambda i: (i, 0)),
            pl.BlockSpec((_D, _D), lambda i: (0, 0)),
            pl.BlockSpec((1, _D), lambda i: (0, 0)),
            pl.BlockSpec((16, _D), lambda i: (0, 0)),
        ],
        out_specs=[
            pl.BlockSpec((nb, _D), lambda i: (i, 0)),
            pl.BlockSpec((1, _D), lambda i: (0, 0)),
            pl.BlockSpec((1, _D), lambda i: (0, 0)),
        ],
        out_shape=[
            jax.ShapeDtypeStruct((_N, _D), _F32),
            jax.ShapeDtypeStruct((1, _D), _F32),
            jax.ShapeDtypeStruct((1, _D), _F32),
        ],
    )(wv, z16, v, wov, bov, bmz)


def _bnffn_body(x_ref, m_ref, r_ref, g_ref, b_ref, w1_ref, b1_ref, w2_ref,
                b2_ref, y_ref, ssum_ref, ssq_ref):
    xn = (x_ref[...].astype(_F32) - m_ref[...]) * r_ref[...] * g_ref[...] \
        + b_ref[...]
    h = jnp.maximum(
        jnp.dot(xn.astype(_BF16), w1_ref[...],
                preferred_element_type=_F32) + b1_ref[...], 0.0)
    y = xn + jnp.dot(h.astype(_BF16), w2_ref[...],
                     preferred_element_type=_F32) + b2_ref[...]
    y_ref[...] = y.astype(y_ref.dtype)

    @pl.when(pl.program_id(0) == 0)
    def _():
        ssum_ref[...] = jnp.zeros_like(ssum_ref)
        ssq_ref[...] = jnp.zeros_like(ssq_ref)

    ssum_ref[...] += jnp.sum(y, axis=0, keepdims=True)
    ssq_ref[...] += jnp.sum(y * y, axis=0, keepdims=True)


def _bnffn_call(x, m, r, g, b, w1, b1, w2, b2, rows, rb, odt=_F32):
    grid = (rows // rb,)
    vec = pl.BlockSpec((1, _D), lambda i: (0, 0))
    return pl.pallas_call(
        _bnffn_body,
        grid=grid,
        in_specs=[
            pl.BlockSpec((rb, _D), lambda i: (i, 0)),
            vec, vec, vec, vec,
            pl.BlockSpec((_D, 2 * _D), lambda i: (0, 0)),
            pl.BlockSpec((1, 2 * _D), lambda i: (0, 0)),
            pl.BlockSpec((2 * _D, _D), lambda i: (0, 0)),
            vec,
        ],
        out_specs=[
            pl.BlockSpec((rb, _D), lambda i: (i, 0)),
            pl.BlockSpec((1, _D), lambda i: (0, 0)),
            pl.BlockSpec((1, _D), lambda i: (0, 0)),
        ],
        out_shape=[
            jax.ShapeDtypeStruct((rows, _D), odt),
            jax.ShapeDtypeStruct((1, _D), _F32),
            jax.ShapeDtypeStruct((1, _D), _F32),
        ],
    )(x, m, r, g, b, w1, b1, w2, b2)


def _bnapply_body(x_ref, m_ref, r_ref, g_ref, b_ref, y_ref):
    y_ref[...] = (x_ref[...].astype(_F32) - m_ref[...]) * r_ref[...] \
        * g_ref[...] + b_ref[...]


def _bnapply_call(x, m, r, g, b, rows, rb):
    grid = (rows // rb,)
    vec = pl.BlockSpec((1, _D), lambda i: (0, 0))
    return pl.pallas_call(
        _bnapply_body,
        grid=grid,
        in_specs=[pl.BlockSpec((rb, _D), lambda i: (i, 0)), vec, vec, vec,
                  vec],
        out_specs=pl.BlockSpec((rb, _D), lambda i: (i, 0)),
        out_shape=jax.ShapeDtypeStruct((rows, _D), _F32),
    )(x, m, r, g, b)


_NC = 2
_NS = 16
_NW = _NC * _NS
_GC = 128
_DP = 128
_NCH = _E // _GC
_GJ = _NCH // _NW
_GT = _NCH - _GJ * _NW
_SC = 128
_SJ = _NCH // _NS
_ST = _NCH - _SJ * _NS
_NP = 10112
_RPS = _NP // _NS
_ZH = _NP // _NC
_ZP = 5120
_ZRS = _ZP // _NS


def _gather3_build():
    mesh = plsc.VectorSubcoreMesh(core_axis_name="c", subcore_axis_name="s", num_cores=_NC, num_subcores=_NS)

    @functools.partial(
        pl.kernel,
        out_type=(
            jax.ShapeDtypeStruct((_E, _DP), _F32),
            jax.ShapeDtypeStruct((_E, _DP), _F32),
            jax.ShapeDtypeStruct((_E, _DP), _F32),
        ),
        mesh=mesh,
        scratch_types=[
            pltpu.VMEM((_GC,), jnp.int32),
            pltpu.VMEM((_GC,), jnp.int32),
            pltpu.VMEM((_GC, _DP), _F32),
            pltpu.VMEM((_GC, _DP), _F32),
            pltpu.VMEM((_GC, _DP), _F32),
            pltpu.SemaphoreType.DMA,
        ],
    )
    def gather3(ktab, qtab, vtab, src, dst, ok, oq, ov, src_v, dst_v, bk, bq,
                bv, sem):
        wid = lax.axis_index("s") * _NC + lax.axis_index("c")

        def do_chunk(off):
            pltpu.sync_copy(src.at[pl.ds(off, _GC)], src_v)
            pltpu.sync_copy(dst.at[pl.ds(off, _GC)], dst_v)
            ck = pltpu.async_copy(ktab.at[src_v], bk, sem)
            cq = pltpu.async_copy(qtab.at[dst_v], bq, sem)
            cv = pltpu.async_copy(vtab.at[src_v], bv, sem)
            ck.wait()
            cq.wait()
            cv.wait()
            pltpu.sync_copy(bk, ok.at[pl.ds(off, _GC)])
            pltpu.sync_copy(bq, oq.at[pl.ds(off, _GC)])
            pltpu.sync_copy(bv, ov.at[pl.ds(off, _GC)])

        def body(j, carry):
            do_chunk((wid + _NW * j) * _GC)
            return carry

        lax.fori_loop(0, _GJ, body, 0)

        @pl.when(wid < _GT)
        def _():
            do_chunk((_GJ * _NW + wid) * _GC)

    return gather3


def _scatter_build():
    mesh = plsc.VectorSubcoreMesh(core_axis_name="c", subcore_axis_name="s", num_cores=_NC, num_subcores=_NS)

    @functools.partial(
        pl.kernel,
        out_type=(
            jax.ShapeDtypeStruct((_NP, _D), _F32),
            jax.ShapeDtypeStruct((_NC * _ZP, 128), _F32),
        ),
        mesh=mesh,
        scratch_types=[
            pltpu.VMEM((_SC,), jnp.int32),
            pltpu.VMEM((_SC,), jnp.int32),
            pltpu.VMEM((_SC, 128), _F32),
            pltpu.VMEM((_SC, 128), _F32),
            pltpu.VMEM((_SC * 16,), _F32),
            pltpu.SemaphoreType.DMA,
            pltpu.SemaphoreType.DMA,
            pltpu.VMEM_SHARED((_NP, 128), _F32),
        ],
    )
    def scatter(ev, s16, dst, zrows, owv, oz, dst_v, dst_v1, evb,
                evb1, sb, sem0, sem1, acc):
        zb = evb1
        dstc_v = dst_v1
        cid = lax.axis_index("c")
        sid = lax.axis_index("s")
        col0 = cid * 128
        node0 = cid * _ZH

        pltpu.sync_copy(zrows, acc.at[pl.ds(sid * _RPS, _RPS)])
        plsc.subcore_barrier()

        def off_of(j):
            return (sid + _NS * j) * _SC

        def issue_in(off, dv, ebuf, sm):
            pltpu.async_copy(dst.at[pl.ds(off, _SC)], dv, sm)
            pltpu.async_copy(ev.at[pl.ds(off, _SC), pl.ds(col0, 128)], ebuf,
                             sm)

        def drain_in(off, dv, ebuf, sm):
            pltpu.make_async_copy(dst.at[pl.ds(off, _SC)], dv, sm).wait()
            pltpu.make_async_copy(ev.at[pl.ds(off, _SC), pl.ds(col0, 128)],
                                  ebuf, sm).wait()

        issue_in(off_of(0), dst_v, evb, sem0)

        def body_ev(i, carry):
            j0 = 2 * i
            j1 = j0 + 1
            drain_in(off_of(j0), dst_v, evb, sem0)
            issue_in(off_of(j1), dst_v1, evb1, sem1)
            pltpu.sync_copy(evb, acc.at[dst_v], add=True)
            drain_in(off_of(j1), dst_v1, evb1, sem1)

            @pl.when(j1 + 1 < _SJ)
            def _():
                issue_in(off_of(j1 + 1), dst_v, evb, sem0)

            pltpu.sync_copy(evb1, acc.at[dst_v1], add=True)
            return carry

        lax.fori_loop(0, _SJ // 2, body_ev, 0)

        @pl.when(sid < _ST)
        def _():
            off = (_SJ * _NS + sid) * _SC
            pltpu.sync_copy(dst.at[pl.ds(off, _SC)], dst_v)
            pltpu.sync_copy(ev.at[pl.ds(off, _SC), pl.ds(col0, 128)], evb)
            pltpu.sync_copy(evb, acc.at[dst_v], add=True)
        plsc.subcore_barrier()
        r0 = sid * _RPS
        pltpu.sync_copy(acc.at[pl.ds(r0, _RPS)],
                        owv.at[pl.ds(r0, _RPS), pl.ds(col0, 128)])
        plsc.subcore_barrier()

        pltpu.sync_copy(zrows.at[pl.ds(0, _ZRS)],
                        acc.at[pl.ds(sid * _ZRS, _ZRS)])
        plsc.subcore_barrier()

        pltpu.sync_copy(zrows.at[pl.ds(0, _SC)], zb)

        def chunk_z(off):
            pltpu.sync_copy(dst.at[pl.ds(off, _SC)], dst_v)
            pltpu.sync_copy(s16.at[pl.ds(off * 16, _SC * 16)], sb)
            for t in range(_SC):
                zb[t, pl.ds(0, 16)] = sb[pl.ds(t * 16, 16)]
            for t in range(_SC // 16):
                iv = dst_v[pl.ds(t * 16, 16)]
                rel = iv - node0
                good = (rel >= 0) & (rel < _ZH)
                dstc_v[pl.ds(t * 16, 16)] = jnp.where(good, rel, _ZH)
            pltpu.sync_copy(zb, acc.at[dstc_v], add=True)

        def body_z(j, carry):
            chunk_z((sid + _NS * j) * _SC)
            return carry

        lax.fori_loop(0, _SJ, body_z, 0)

        @pl.when(sid < _ST)
        def _():
            chunk_z((_SJ * _NS + sid) * _SC)
        plsc.subcore_barrier()
        rz = sid * _ZRS
        pltpu.sync_copy(acc.at[pl.ds(rz, _ZRS)],
                        oz.at[pl.ds(cid * _ZP + rz, _ZRS)])

    return scatter


_GATHER3 = None
_SCATTER = None


def _gather3_run(k, q, vv, src, dst):
    global _GATHER3
    if _GATHER3 is None:
        _GATHER3 = _gather3_build()
    return _GATHER3(k, q, vv, src, dst)


def _scatter_run(ev, s16, dst, zrows):
    global _SCATTER
    if _SCATTER is None:
        _SCATTER = _scatter_build()
    return _SCATTER(ev, s16, dst, zrows)


_SMASK = (np.arange(_D)[:, None] // _DH ==
          np.arange(_H)[None, :]).astype(np.float32)
_BMASK = (np.arange(_D)[None, :] // _DH ==
          np.arange(_H)[:, None]).astype(np.float32)
_BMZ = np.concatenate([_BMASK, np.zeros((8, _D), np.float32)], axis=0)


def _row(x):
    return x.reshape(1, -1)


def kernel(v, e, edge_index, WQ, WK, WV, We, WOv, bOv, WOe, bOe, W1v, b1v,
           W2v, b2v, W1e, b1e, W2e, b2e, g1v, be1v, g1e, be1e, g2v, be2v,
           g2e, be2e):
    src = edge_index[0]
    dst = edge_index[1]
    wqkv = jnp.concatenate([WQ, WK, WV], axis=1)
    we16 = We.astype(_BF16)
    woe16 = WOe.astype(_BF16)
    wov16 = WOv.astype(_BF16)

    q, k, vv = _qkv_call(v, wqkv)

    ksrc, qdst, vsrc = _gather3_run(k, q, vv, src, dst)

    e1, ev, s16, s1, q1 = _edge_a_call(e, ksrc, qdst, vsrc, we16, woe16,
                                       _row(bOe), _SMASK, _BMASK)

    zrows = jnp.zeros((_RPS, 128), _F32)
    wv_pad, oz = _scatter_run(ev, s16.reshape(-1), dst, zrows)
    wv = wv_pad[:_N]
    z16 = jnp.concatenate(
        [oz[:_ZH, :16], oz[_ZP:_ZP + _N - _ZH, :16]], axis=0)

    m1 = s1 / _E
    r1 = lax.rsqrt(q1 / _E - m1 * m1 + 1e-5)
    e2, s2, q2 = _bnffn_call(e1, m1, r1, _row(g1e), _row(be1e), W1e.astype(_BF16),
                             _row(b1e), W2e, _row(b2e), _E, 1000)
    m2 = s2 / _E
    r2 = lax.rsqrt(q2 / _E - m2 * m2 + 1e-5)
    out_e = _bnapply_call(e2, m2, r2, _row(g2e), _row(be2e), _E, 1000)

    v1, sv1, qv1 = _vatt_call(wv, z16, v, wov16, _row(bOv), _BMZ)
    mv1 = sv1 / _N
    rv1 = lax.rsqrt(qv1 / _N - mv1 * mv1 + 1e-5)
    v2, sv2, qv2 = _bnffn_call(v1, mv1, rv1, _row(g1v), _row(be1v), W1v.astype(_BF16),
                               _row(b1v), W2v.astype(_BF16), _row(b2v), _N, 400)
    mv2 = sv2 / _N
    rv2 = lax.rsqrt(qv2 / _N - mv2 * mv2 + 1e-5)
    out_v = _bnapply_call(v2, mv2, rv2, _row(g2v), _row(be2v), _N, 400)

    return (out_v, out_e)


# speedup vs baseline: 3.4379x; 1.0457x over previous
import functools
import numpy as np
import jax
import jax.numpy as jnp
from jax import lax
from jax.experimental import pallas as pl
from jax.experimental.pallas import tpu as pltpu
from jax.experimental.pallas import tpu_sc as plsc

_N = 10000
_E = 160000
_D = 256
_H = 8
_DH = 32
_F32 = jnp.float32
_BF16 = jnp.bfloat16


def _pack_cols(y):
    lo = lax.bitcast_convert_type(y[:, :128].astype(_BF16),
                                  jnp.uint16).astype(jnp.uint32)
    hi = lax.bitcast_convert_type(y[:, 128:].astype(_BF16),
                                  jnp.uint16).astype(jnp.uint32)
    return lax.bitcast_convert_type(lo | (hi << 16), _F32)


def _unpack_cols(x):
    xi = lax.bitcast_convert_type(x, jnp.uint32)
    lo = lax.bitcast_convert_type(xi << 16, _F32)
    hi = lax.bitcast_convert_type(xi & jnp.uint32(0xFFFF0000), _F32)
    return jnp.concatenate([lo, hi], axis=1)


def _qkv_body(v_ref, w_ref, q_ref, k_ref, vv_ref):
    y = jnp.dot(v_ref[...], w_ref[...], preferred_element_type=_F32)
    q_ref[...] = _pack_cols(y[:, :_D])
    k_ref[...] = _pack_cols(y[:, _D:2 * _D])
    vv_ref[...] = _pack_cols(y[:, 2 * _D:])


def _qkv_call(v, wqkv):
    nb = 400
    grid = (_N // nb,)
    return pl.pallas_call(
        _qkv_body,
        grid=grid,
        in_specs=[
            pl.BlockSpec((nb, _D), lambda i: (i, 0)),
            pl.BlockSpec((_D, 3 * _D), lambda i: (0, 0)),
        ],
        out_specs=[
            pl.BlockSpec((nb, 128), lambda i: (i, 0)),
            pl.BlockSpec((nb, 128), lambda i: (i, 0)),
            pl.BlockSpec((nb, 128), lambda i: (i, 0)),
        ],
        out_shape=[jax.ShapeDtypeStruct((_N, 128), _F32)] * 3,
    )(v, wqkv)


def _edge_a_body(e_ref, ks_ref, qd_ref, vs_ref, we_ref, woe_ref, boe_ref,
                 smask_ref, bmask_ref, e1_ref, ev_ref, s16_ref, ssum_ref,
                 ssq_ref):
    eb = e_ref[...]
    pe = jnp.dot(eb.astype(_BF16), we_ref[...], preferred_element_type=_F32)
    kq = _unpack_cols(ks_ref[...]) * _unpack_cols(qd_ref[...])
    score = kq * pe * np.float32(1.0 / np.sqrt(_DH))
    shead = jnp.dot(score, smask_ref[...], preferred_element_type=_F32)
    sexp = jnp.exp(jnp.clip(shead, -5.0, 5.0))
    e1 = eb + jnp.dot(score.astype(_BF16), woe_ref[...],
                      preferred_element_type=_F32) + boe_ref[...]
    e1_ref[...] = e1.astype(e1_ref.dtype)
    ev_ref[...] = _unpack_cols(vs_ref[...]) * jnp.dot(
        sexp, bmask_ref[...], preferred_element_type=_F32)
    s16_ref[...] = jnp.concatenate(
        [sexp, jnp.zeros((sexp.shape[0], 8), _F32)], axis=1)

    @pl.when(pl.program_id(0) == 0)
    def _():
        ssum_ref[...] = jnp.zeros_like(ssum_ref)
        ssq_ref[...] = jnp.zeros_like(ssq_ref)

    ssum_ref[...] += jnp.sum(e1, axis=0, keepdims=True)
    ssq_ref[...] += jnp.sum(e1 * e1, axis=0, keepdims=True)


def _edge_a_call(e, ksrc, qdst, vsrc, we, woe, boe, smask, bmask):
    eb = 1000
    grid = (_E // eb,)
    big = pl.BlockSpec((eb, _D), lambda i: (i, 0))
    pk = pl.BlockSpec((eb, 128), lambda i: (i, 0))
    return pl.pallas_call(
        _edge_a_body,
        grid=grid,
        in_specs=[
            big, pk, pk, pk,
            pl.BlockSpec((_D, _D), lambda i: (0, 0)),
            pl.BlockSpec((_D, _D), lambda i: (0, 0)),
            pl.BlockSpec((1, _D), lambda i: (0, 0)),
            pl.BlockSpec((_D, _H), lambda i: (0, 0)),
            pl.BlockSpec((_H, _D), lambda i: (0, 0)),
        ],
        out_specs=[
            big, big,
            pl.BlockSpec((eb, 16), lambda i: (i, 0)),
            pl.BlockSpec((1, _D), lambda i: (0, 0)),
            pl.BlockSpec((1, _D), lambda i: (0, 0)),
        ],
        out_shape=[
            jax.ShapeDtypeStruct((_E, _D), _BF16),
            jax.ShapeDtypeStruct((_E, _D), _F32),
            jax.ShapeDtypeStruct((_E, 16), _F32),
            jax.ShapeDtypeStruct((1, _D), _F32),
            jax.ShapeDtypeStruct((1, _D), _F32),
        ],
    )(e, ksrc, qdst, vsrc, we, woe, boe, smask, bmask)


def _vatt_body(wv_ref, z_ref, v_ref, wov_ref, bov_ref, bmz_ref, v1_ref,
               ssum_ref, ssq_ref):
    zb = jnp.dot(z_ref[...], bmz_ref[...], preferred_element_type=_F32)
    vatt = wv_ref[...] / (zb + 1e-6)
    v1 = v_ref[...] + jnp.dot(vatt.astype(_BF16), wov_ref[...],
                              preferred_element_type=_F32) + bov_ref[...]
    v1_ref[...] = v1

    @pl.when(pl.program_id(0) == 0)
    def _():
        ssum_ref[...] = jnp.zeros_like(ssum_ref)
        ssq_ref[...] = jnp.zeros_like(ssq_ref)

    ssum_ref[...] += jnp.sum(v1, axis=0, keepdims=True)
    ssq_ref[...] += jnp.sum(v1 * v1, axis=0, keepdims=True)


def _vatt_call(wv, z16, v, wov, bov, bmz):
    nb = 400
    grid = (_N // nb,)
    return pl.pallas_call(
        _vatt_body,
        grid=grid,
        in_specs=[
            pl.BlockSpec((nb, _D), lambda i: (i, 0)),
            pl.BlockSpec((nb, 16), lambda i: (i, 0)),
            pl.BlockSpec((nb, _D), lambda i: (i, 0)),
            pl.BlockSpec((_D, _D), lambda i: (0, 0)),
            pl.BlockSpec((1, _D), lambda i: (0, 0)),
            pl.BlockSpec((16, _D), lambda i: (0, 0)),
        ],
        out_specs=[
            pl.BlockSpec((nb, _D), lambda i: (i, 0)),
            pl.BlockSpec((1, _D), lambda i: (0, 0)),
            pl.BlockSpec((1, _D), lambda i: (0, 0)),
        ],
        out_shape=[
            jax.ShapeDtypeStruct((_N, _D), _F32),
            jax.ShapeDtypeStruct((1, _D), _F32),
            jax.ShapeDtypeStruct((1, _D), _F32),
        ],
    )(wv, z16, v, wov, bov, bmz)


def _bnffn_body(x_ref, m_ref, r_ref, g_ref, b_ref, w1_ref, b1_ref, w2_ref,
                b2_ref, y_ref, ssum_ref, ssq_ref):
    xn = (x_ref[...].astype(_F32) - m_ref[...]) * r_ref[...] * g_ref[...] \
        + b_ref[...]
    h = jnp.maximum(
        jnp.dot(xn.astype(_BF16), w1_ref[...],
                preferred_element_type=_F32) + b1_ref[...], 0.0)
    y = xn + jnp.dot(h.astype(_BF16), w2_ref[...],
                     preferred_element_type=_F32) + b2_ref[...]
    y_ref[...] = y.astype(y_ref.dtype)

    @pl.when(pl.program_id(0) == 0)
    def _():
        ssum_ref[...] = jnp.zeros_like(ssum_ref)
        ssq_ref[...] = jnp.zeros_like(ssq_ref)

    ssum_ref[...] += jnp.sum(y, axis=0, keepdims=True)
    ssq_ref[...] += jnp.sum(y * y, axis=0, keepdims=True)


def _bnffn_call(x, m, r, g, b, w1, b1, w2, b2, rows, rb, odt=_F32):
    grid = (rows // rb,)
    vec = pl.BlockSpec((1, _D), lambda i: (0, 0))
    return pl.pallas_call(
        _bnffn_body,
        grid=grid,
        in_specs=[
            pl.BlockSpec((rb, _D), lambda i: (i, 0)),
            vec, vec, vec, vec,
            pl.BlockSpec((_D, 2 * _D), lambda i: (0, 0)),
            pl.BlockSpec((1, 2 * _D), lambda i: (0, 0)),
            pl.BlockSpec((2 * _D, _D), lambda i: (0, 0)),
            vec,
        ],
        out_specs=[
            pl.BlockSpec((rb, _D), lambda i: (i, 0)),
            pl.BlockSpec((1, _D), lambda i: (0, 0)),
            pl.BlockSpec((1, _D), lambda i: (0, 0)),
        ],
        out_shape=[
            jax.ShapeDtypeStruct((rows, _D), odt),
            jax.ShapeDtypeStruct((1, _D), _F32),
            jax.ShapeDtypeStruct((1, _D), _F32),
        ],
    )(x, m, r, g, b, w1, b1, w2, b2)


def _bnapply_body(x_ref, m_ref, r_ref, g_ref, b_ref, y_ref):
    y_ref[...] = (x_ref[...].astype(_F32) - m_ref[...]) * r_ref[...] \
        * g_ref[...] + b_ref[...]


def _bnapply_call(x, m, r, g, b, rows, rb):
    grid = (rows // rb,)
    vec = pl.BlockSpec((1, _D), lambda i: (0, 0))
    return pl.pallas_call(
        _bnapply_body,
        grid=grid,
        in_specs=[pl.BlockSpec((rb, _D), lambda i: (i, 0)), vec, vec, vec,
                  vec],
        out_specs=pl.BlockSpec((rb, _D), lambda i: (i, 0)),
        out_shape=jax.ShapeDtypeStruct((rows, _D), _F32),
    )(x, m, r, g, b)


_NC = 2
_NS = 16
_NW = _NC * _NS
_GC = 128
_DP = 128
_NCH = _E // _GC
_GJ = _NCH // _NW
_GT = _NCH - _GJ * _NW
_SC = 128
_SJ = _NCH // _NS
_ST = _NCH - _SJ * _NS
_NP = 10112
_RPS = _NP // _NS
_ZH = _NP // _NC
_ZP = 5120
_ZRS = _ZP // _NS


def _gather3_build():
    mesh = plsc.VectorSubcoreMesh(core_axis_name="c", subcore_axis_name="s", num_cores=_NC, num_subcores=_NS)

    @functools.partial(
        pl.kernel,
        out_type=(
            jax.ShapeDtypeStruct((_E, _DP), _F32),
            jax.ShapeDtypeStruct((_E, _DP), _F32),
            jax.ShapeDtypeStruct((_E, _DP), _F32),
        ),
        mesh=mesh,
        scratch_types=[
            pltpu.VMEM((_GC,), jnp.int32),
            pltpu.VMEM((_GC,), jnp.int32),
            pltpu.VMEM((_GC,), jnp.int32),
            pltpu.VMEM((_GC,), jnp.int32),
            pltpu.VMEM((_GC, _DP), _F32),
            pltpu.VMEM((_GC, _DP), _F32),
            pltpu.VMEM((_GC, _DP), _F32),
            pltpu.VMEM((_GC, _DP), _F32),
            pltpu.VMEM((_GC, _DP), _F32),
            pltpu.VMEM((_GC, _DP), _F32),
            pltpu.SemaphoreType.DMA,
            pltpu.SemaphoreType.DMA,
        ],
    )
    def gather3(ktab, qtab, vtab, src, dst, ok, oq, ov, sv0, dv0, sv1, dv1,
                bk0, bq0, bv0, bk1, bq1, bv1, sem0, sem1):
        wid = lax.axis_index("s") * _NC + lax.axis_index("c")

        def off_of(j):
            return (wid + _NW * j) * _GC

        def issue(off, sv, dv, bk, bq, bv, sm):
            pltpu.sync_copy(src.at[pl.ds(off, _GC)], sv)
            pltpu.sync_copy(dst.at[pl.ds(off, _GC)], dv)
            pltpu.async_copy(ktab.at[sv], bk, sm)
            pltpu.async_copy(qtab.at[dv], bq, sm)
            pltpu.async_copy(vtab.at[sv], bv, sm)

        def drain_wb(off, sv, dv, bk, bq, bv, sm):
            pltpu.make_async_copy(ktab.at[sv], bk, sm).wait()
            pltpu.make_async_copy(qtab.at[dv], bq, sm).wait()
            pltpu.make_async_copy(vtab.at[sv], bv, sm).wait()
            pltpu.sync_copy(bk, ok.at[pl.ds(off, _GC)])
            pltpu.sync_copy(bq, oq.at[pl.ds(off, _GC)])
            pltpu.sync_copy(bv, ov.at[pl.ds(off, _GC)])

        issue(off_of(0), sv0, dv0, bk0, bq0, bv0, sem0)

        def body(i, carry):
            j0 = 2 * i
            j1 = j0 + 1
            issue(off_of(j1), sv1, dv1, bk1, bq1, bv1, sem1)
            drain_wb(off_of(j0), sv0, dv0, bk0, bq0, bv0, sem0)

            @pl.when(j1 + 1 < _GJ)
            def _():
                issue(off_of(j1 + 1), sv0, dv0, bk0, bq0, bv0, sem0)

            drain_wb(off_of(j1), sv1, dv1, bk1, bq1, bv1, sem1)
            return carry

        lax.fori_loop(0, _GJ // 2, body, 0)

        drain_wb(off_of(_GJ - 1), sv0, dv0, bk0, bq0, bv0, sem0)

        @pl.when(wid < _GT)
        def _():
            off = (_GJ * _NW + wid) * _GC
            issue(off, sv1, dv1, bk1, bq1, bv1, sem1)
            drain_wb(off, sv1, dv1, bk1, bq1, bv1, sem1)

    return gather3


def _scatter_build():
    mesh = plsc.VectorSubcoreMesh(core_axis_name="c", subcore_axis_name="s", num_cores=_NC, num_subcores=_NS)

    @functools.partial(
        pl.kernel,
        out_type=(
            jax.ShapeDtypeStruct((_NP, _D), _F32),
            jax.ShapeDtypeStruct((_NC * _ZP, 128), _F32),
        ),
        mesh=mesh,
        scratch_types=[
            pltpu.VMEM((_SC,), jnp.int32),
            pltpu.VMEM((_SC,), jnp.int32),
            pltpu.VMEM((_SC, 128), _F32),
            pltpu.VMEM((_SC, 128), _F32),
            pltpu.VMEM((_SC * 16,), _F32),
            pltpu.SemaphoreType.DMA,
            pltpu.SemaphoreType.DMA,
            pltpu.VMEM_SHARED((_NP, 128), _F32),
        ],
    )
    def scatter(ev, s16, dst, zrows, owv, oz, dst_v, dst_v1, evb,
                evb1, sb, sem0, sem1, acc):
        zb = evb1
        dstc_v = dst_v1
        cid = lax.axis_index("c")
        sid = lax.axis_index("s")
        col0 = cid * 128
        node0 = cid * _ZH

        pltpu.sync_copy(zrows, acc.at[pl.ds(sid * _RPS, _RPS)])
        plsc.subcore_barrier()

        def off_of(j):
            return (sid + _NS * j) * _SC

        def issue_in(off, dv, ebuf, sm):
            pltpu.async_copy(dst.at[pl.ds(off, _SC)], dv, sm)
            pltpu.async_copy(ev.at[pl.ds(off, _SC), pl.ds(col0, 128)], ebuf,
                             sm)

        def drain_in(off, dv, ebuf, sm):
            pltpu.make_async_copy(dst.at[pl.ds(off, _SC)], dv, sm).wait()
            pltpu.make_async_copy(ev.at[pl.ds(off, _SC), pl.ds(col0, 128)],
                                  ebuf, sm).wait()

        issue_in(off_of(0), dst_v, evb, sem0)

        def body_ev(i, carry):
            j0 = 2 * i
            j1 = j0 + 1
            drain_in(off_of(j0), dst_v, evb, sem0)
            issue_in(off_of(j1), dst_v1, evb1, sem1)
            pltpu.sync_copy(evb, acc.at[dst_v], add=True)
            drain_in(off_of(j1), dst_v1, evb1, sem1)

            @pl.when(j1 + 1 < _SJ)
            def _():
                issue_in(off_of(j1 + 1), dst_v, evb, sem0)

            pltpu.sync_copy(evb1, acc.at[dst_v1], add=True)
            return carry

        lax.fori_loop(0, _SJ // 2, body_ev, 0)

        @pl.when(sid < _ST)
        def _():
            off = (_SJ * _NS + sid) * _SC
            pltpu.sync_copy(dst.at[pl.ds(off, _SC)], dst_v)
            pltpu.sync_copy(ev.at[pl.ds(off, _SC), pl.ds(col0, 128)], evb)
            pltpu.sync_copy(evb, acc.at[dst_v], add=True)
        plsc.subcore_barrier()
        r0 = sid * _RPS
        pltpu.sync_copy(acc.at[pl.ds(r0, _RPS)],
                        owv.at[pl.ds(r0, _RPS), pl.ds(col0, 128)])
        plsc.subcore_barrier()

        pltpu.sync_copy(zrows.at[pl.ds(0, _ZRS)],
                        acc.at[pl.ds(sid * _ZRS, _ZRS)])
        plsc.subcore_barrier()

        pltpu.sync_copy(zrows.at[pl.ds(0, _SC)], zb)

        def chunk_z(off):
            pltpu.sync_copy(dst.at[pl.ds(off, _SC)], dst_v)
            pltpu.sync_copy(s16.at[pl.ds(off * 16, _SC * 16)], sb)
            for t in range(_SC):
                zb[t, pl.ds(0, 16)] = sb[pl.ds(t * 16, 16)]
            for t in range(_SC // 16):
                iv = dst_v[pl.ds(t * 16, 16)]
                rel = iv - node0
                good = (rel >= 0) & (rel < _ZH)
                dstc_v[pl.ds(t * 16, 16)] = jnp.where(good, rel, _ZH)
            pltpu.sync_copy(zb, acc.at[dstc_v], add=True)

        def body_z(j, carry):
            chunk_z((sid + _NS * j) * _SC)
            return carry

        lax.fori_loop(0, _SJ, body_z, 0)

        @pl.when(sid < _ST)
        def _():
            chunk_z((_SJ * _NS + sid) * _SC)
        plsc.subcore_barrier()
        rz = sid * _ZRS
        pltpu.sync_copy(acc.at[pl.ds(rz, _ZRS)],
                        oz.at[pl.ds(cid * _ZP + rz, _ZRS)])

    return scatter


_GATHER3 = None
_SCATTER = None


def _gather3_run(k, q, vv, src, dst):
    global _GATHER3
    if _GATHER3 is None:
        _GATHER3 = _gather3_build()
    return _GATHER3(k, q, vv, src, dst)


def _scatter_run(ev, s16, dst, zrows):
    global _SCATTER
    if _SCATTER is None:
        _SCATTER = _scatter_build()
    return _SCATTER(ev, s16, dst, zrows)


_SMASK = (np.arange(_D)[:, None] // _DH ==
          np.arange(_H)[None, :]).astype(np.float32)
_BMASK = (np.arange(_D)[None, :] // _DH ==
          np.arange(_H)[:, None]).astype(np.float32)
_BMZ = np.concatenate([_BMASK, np.zeros((8, _D), np.float32)], axis=0)


def _row(x):
    return x.reshape(1, -1)


def kernel(v, e, edge_index, WQ, WK, WV, We, WOv, bOv, WOe, bOe, W1v, b1v,
           W2v, b2v, W1e, b1e, W2e, b2e, g1v, be1v, g1e, be1e, g2v, be2v,
           g2e, be2e):
    src = edge_index[0]
    dst = edge_index[1]
    wqkv = jnp.concatenate([WQ, WK, WV], axis=1)
    we16 = We.astype(_BF16)
    woe16 = WOe.astype(_BF16)
    wov16 = WOv.astype(_BF16)

    q, k, vv = _qkv_call(v, wqkv)

    ksrc, qdst, vsrc = _gather3_run(k, q, vv, src, dst)

    e1, ev, s16, s1, q1 = _edge_a_call(e, ksrc, qdst, vsrc, we16, woe16,
                                       _row(bOe), _SMASK, _BMASK)

    zrows = jnp.zeros((_RPS, 128), _F32)
    wv_pad, oz = _scatter_run(ev, s16.reshape(-1), dst, zrows)
    wv = wv_pad[:_N]
    z16 = jnp.concatenate(
        [oz[:_ZH, :16], oz[_ZP:_ZP + _N - _ZH, :16]], axis=0)

    m1 = s1 / _E
    r1 = lax.rsqrt(q1 / _E - m1 * m1 + 1e-5)
    e2, s2, q2 = _bnffn_call(e1, m1, r1, _row(g1e), _row(be1e), W1e.astype(_BF16),
                             _row(b1e), W2e, _row(b2e), _E, 1000)
    m2 = s2 / _E
    r2 = lax.rsqrt(q2 / _E - m2 * m2 + 1e-5)
    out_e = _bnapply_call(e2, m2, r2, _row(g2e), _row(be2e), _E, 1000)

    v1, sv1, qv1 = _vatt_call(wv, z16, v, wov16, _row(bOv), _BMZ)
    mv1 = sv1 / _N
    rv1 = lax.rsqrt(qv1 / _N - mv1 * mv1 + 1e-5)
    v2, sv2, qv2 = _bnffn_call(v1, mv1, rv1, _row(g1v), _row(be1v), W1v.astype(_BF16),
                               _row(b1v), W2v.astype(_BF16), _row(b2v), _N, 400)
    mv2 = sv2 / _N
    rv2 = lax.rsqrt(qv2 / _N - mv2 * mv2 + 1e-5)
    out_v = _bnapply_call(v2, mv2, rv2, _row(g2v), _row(be2v), _N, 400)

    return (out_v, out_e)


# speedup vs baseline: 3.4381x; 1.0001x over previous
import functools
import numpy as np
import jax
import jax.numpy as jnp
from jax import lax
from jax.experimental import pallas as pl
from jax.experimental.pallas import tpu as pltpu
from jax.experimental.pallas import tpu_sc as plsc

_N = 10000
_E = 160000
_D = 256
_H = 8
_DH = 32
_F32 = jnp.float32
_BF16 = jnp.bfloat16


def _pack_cols(y):
    lo = lax.bitcast_convert_type(y[:, :128].astype(_BF16),
                                  jnp.uint16).astype(jnp.uint32)
    hi = lax.bitcast_convert_type(y[:, 128:].astype(_BF16),
                                  jnp.uint16).astype(jnp.uint32)
    return lax.bitcast_convert_type(lo | (hi << 16), _F32)


def _unpack_cols(x):
    xi = lax.bitcast_convert_type(x, jnp.uint32)
    lo = lax.bitcast_convert_type(xi << 16, _F32)
    hi = lax.bitcast_convert_type(xi & jnp.uint32(0xFFFF0000), _F32)
    return jnp.concatenate([lo, hi], axis=1)


def _qkv_body(v_ref, w_ref, q_ref, k_ref, vv_ref):
    y = jnp.dot(v_ref[...], w_ref[...], preferred_element_type=_F32)
    q_ref[...] = _pack_cols(y[:, :_D])
    k_ref[...] = _pack_cols(y[:, _D:2 * _D])
    vv_ref[...] = _pack_cols(y[:, 2 * _D:])


def _qkv_call(v, wqkv):
    nb = 400
    grid = (_N // nb,)
    return pl.pallas_call(
        _qkv_body,
        grid=grid,
        in_specs=[
            pl.BlockSpec((nb, _D), lambda i: (i, 0)),
            pl.BlockSpec((_D, 3 * _D), lambda i: (0, 0)),
        ],
        out_specs=[
            pl.BlockSpec((nb, 128), lambda i: (i, 0)),
            pl.BlockSpec((nb, 128), lambda i: (i, 0)),
            pl.BlockSpec((nb, 128), lambda i: (i, 0)),
        ],
        out_shape=[jax.ShapeDtypeStruct((_N, 128), _F32)] * 3,
    )(v, wqkv)


def _edge_a_body(e_ref, ks_ref, qd_ref, vs_ref, we_ref, woe_ref, boe_ref,
                 smask_ref, bmask_ref, e1_ref, ev_ref, s16_ref, ssum_ref,
                 ssq_ref):
    eb = e_ref[...]
    pe = jnp.dot(eb.astype(_BF16), we_ref[...], preferred_element_type=_F32)
    kq = _unpack_cols(ks_ref[...]) * _unpack_cols(qd_ref[...])
    score = kq * pe * np.float32(1.0 / np.sqrt(_DH))
    shead = jnp.dot(score, smask_ref[...], preferred_element_type=_F32)
    sexp = jnp.exp(jnp.clip(shead, -5.0, 5.0))
    e1 = eb + jnp.dot(score.astype(_BF16), woe_ref[...],
                      preferred_element_type=_F32) + boe_ref[...]
    e1_ref[...] = e1.astype(e1_ref.dtype)
    ev_ref[...] = _unpack_cols(vs_ref[...]) * jnp.dot(
        sexp, bmask_ref[...], preferred_element_type=_F32)
    s16_ref[...] = jnp.concatenate(
        [sexp, jnp.zeros((sexp.shape[0], 8), _F32)], axis=1)

    @pl.when(pl.program_id(0) == 0)
    def _():
        ssum_ref[...] = jnp.zeros_like(ssum_ref)
        ssq_ref[...] = jnp.zeros_like(ssq_ref)

    ssum_ref[...] += jnp.sum(e1, axis=0, keepdims=True)
    ssq_ref[...] += jnp.sum(e1 * e1, axis=0, keepdims=True)


def _edge_a_call(e, ksrc, qdst, vsrc, we, woe, boe, smask, bmask):
    eb = 1000
    grid = (_E // eb,)
    big = pl.BlockSpec((eb, _D), lambda i: (i, 0))
    pk = pl.BlockSpec((eb, 128), lambda i: (i, 0))
    return pl.pallas_call(
        _edge_a_body,
        grid=grid,
        in_specs=[
            big, pk, pk, pk,
            pl.BlockSpec((_D, _D), lambda i: (0, 0)),
            pl.BlockSpec((_D, _D), lambda i: (0, 0)),
            pl.BlockSpec((1, _D), lambda i: (0, 0)),
            pl.BlockSpec((_D, _H), lambda i: (0, 0)),
            pl.BlockSpec((_H, _D), lambda i: (0, 0)),
        ],
        out_specs=[
            big, big,
            pl.BlockSpec((eb, 16), lambda i: (i, 0)),
            pl.BlockSpec((1, _D), lambda i: (0, 0)),
            pl.BlockSpec((1, _D), lambda i: (0, 0)),
        ],
        out_shape=[
            jax.ShapeDtypeStruct((_E, _D), _BF16),
            jax.ShapeDtypeStruct((_E, _D), _F32),
            jax.ShapeDtypeStruct((_E, 16), _F32),
            jax.ShapeDtypeStruct((1, _D), _F32),
            jax.ShapeDtypeStruct((1, _D), _F32),
        ],
    )(e, ksrc, qdst, vsrc, we, woe, boe, smask, bmask)


def _vatt_body(wv_ref, z_ref, v_ref, wov_ref, bov_ref, bmz_ref, v1_ref,
               ssum_ref, ssq_ref):
    zb = jnp.dot(z_ref[...], bmz_ref[...], preferred_element_type=_F32)
    vatt = wv_ref[...] / (zb + 1e-6)
    v1 = v_ref[...] + jnp.dot(vatt.astype(_BF16), wov_ref[...],
                              preferred_element_type=_F32) + bov_ref[...]
    v1_ref[...] = v1

    @pl.when(pl.program_id(0) == 0)
    def _():
        ssum_ref[...] = jnp.zeros_like(ssum_ref)
        ssq_ref[...] = jnp.zeros_like(ssq_ref)

    ssum_ref[...] += jnp.sum(v1, axis=0, keepdims=True)
    ssq_ref[...] += jnp.sum(v1 * v1, axis=0, keepdims=True)


def _vatt_call(wv, z16, v, wov, bov, bmz):
    nb = 400
    grid = (_N // nb,)
    return pl.pallas_call(
        _vatt_body,
        grid=grid,
        in_specs=[
            pl.BlockSpec((nb, _D), lambda i: (i, 0)),
            pl.BlockSpec((nb, 16), lambda i: (i, 0)),
            pl.BlockSpec((nb, _D), lambda i: (i, 0)),
            pl.BlockSpec((_D, _D), lambda i: (0, 0)),
            pl.BlockSpec((1, _D), lambda i: (0, 0)),
            pl.BlockSpec((16, _D), lambda i: (0, 0)),
        ],
        out_specs=[
            pl.BlockSpec((nb, _D), lambda i: (i, 0)),
            pl.BlockSpec((1, _D), lambda i: (0, 0)),
            pl.BlockSpec((1, _D), lambda i: (0, 0)),
        ],
        out_shape=[
            jax.ShapeDtypeStruct((_N, _D), _F32),
            jax.ShapeDtypeStruct((1, _D), _F32),
            jax.ShapeDtypeStruct((1, _D), _F32),
        ],
    )(wv, z16, v, wov, bov, bmz)


def _bnffn_body(x_ref, m_ref, r_ref, g_ref, b_ref, w1_ref, b1_ref, w2_ref,
                b2_ref, y_ref, ssum_ref, ssq_ref):
    xn = (x_ref[...].astype(_F32) - m_ref[...]) * r_ref[...] * g_ref[...] \
        + b_ref[...]
    h = jnp.maximum(
        jnp.dot(xn.astype(_BF16), w1_ref[...],
                preferred_element_type=_F32) + b1_ref[...], 0.0)
    y = xn + jnp.dot(h.astype(_BF16), w2_ref[...],
                     preferred_element_type=_F32) + b2_ref[...]
    y_ref[...] = y.astype(y_ref.dtype)

    @pl.when(pl.program_id(0) == 0)
    def _():
        ssum_ref[...] = jnp.zeros_like(ssum_ref)
        ssq_ref[...] = jnp.zeros_like(ssq_ref)

    ssum_ref[...] += jnp.sum(y, axis=0, keepdims=True)
    ssq_ref[...] += jnp.sum(y * y, axis=0, keepdims=True)


def _bnffn_call(x, m, r, g, b, w1, b1, w2, b2, rows, rb, odt=_F32):
    grid = (rows // rb,)
    vec = pl.BlockSpec((1, _D), lambda i: (0, 0))
    return pl.pallas_call(
        _bnffn_body,
        grid=grid,
        in_specs=[
            pl.BlockSpec((rb, _D), lambda i: (i, 0)),
            vec, vec, vec, vec,
            pl.BlockSpec((_D, 2 * _D), lambda i: (0, 0)),
            pl.BlockSpec((1, 2 * _D), lambda i: (0, 0)),
            pl.BlockSpec((2 * _D, _D), lambda i: (0, 0)),
            vec,
        ],
        out_specs=[
            pl.BlockSpec((rb, _D), lambda i: (i, 0)),
            pl.BlockSpec((1, _D), lambda i: (0, 0)),
            pl.BlockSpec((1, _D), lambda i: (0, 0)),
        ],
        out_shape=[
            jax.ShapeDtypeStruct((rows, _D), odt),
            jax.ShapeDtypeStruct((1, _D), _F32),
            jax.ShapeDtypeStruct((1, _D), _F32),
        ],
    )(x, m, r, g, b, w1, b1, w2, b2)


def _bnapply_body(x_ref, m_ref, r_ref, g_ref, b_ref, y_ref):
    y_ref[...] = (x_ref[...].astype(_F32) - m_ref[...]) * r_ref[...] \
        * g_ref[...] + b_ref[...]


def _bnapply_call(x, m, r, g, b, rows, rb):
    grid = (rows // rb,)
    vec = pl.BlockSpec((1, _D), lambda i: (0, 0))
    return pl.pallas_call(
        _bnapply_body,
        grid=grid,
        in_specs=[pl.BlockSpec((rb, _D), lambda i: (i, 0)), vec, vec, vec,
                  vec],
        out_specs=pl.BlockSpec((rb, _D), lambda i: (i, 0)),
        out_shape=jax.ShapeDtypeStruct((rows, _D), _F32),
    )(x, m, r, g, b)


_NC = 2
_NS = 16
_NW = _NC * _NS
_GC = 128
_DP = 128
_NCH = _E // _GC
_GJ = _NCH // _NW
_GT = _NCH - _GJ * _NW
_SC = 128
_SJ = _NCH // _NS
_ST = _NCH - _SJ * _NS
_NP = 10112
_RPS = _NP // _NS
_ZH = _NP // _NC
_ZP = 5120
_ZRS = _ZP // _NS


def _gather3_build():
    mesh = plsc.VectorSubcoreMesh(core_axis_name="c", subcore_axis_name="s", num_cores=_NC, num_subcores=_NS)

    @functools.partial(
        pl.kernel,
        out_type=(
            jax.ShapeDtypeStruct((_E, _DP), _F32),
            jax.ShapeDtypeStruct((_E, _DP), _F32),
            jax.ShapeDtypeStruct((_E, _DP), _F32),
        ),
        mesh=mesh,
        scratch_types=[
            pltpu.VMEM((_GC,), jnp.int32),
            pltpu.VMEM((_GC,), jnp.int32),
            pltpu.VMEM((_GC,), jnp.int32),
            pltpu.VMEM((_GC,), jnp.int32),
            pltpu.VMEM((_GC, _DP), _F32),
            pltpu.VMEM((_GC, _DP), _F32),
            pltpu.VMEM((_GC, _DP), _F32),
            pltpu.VMEM((_GC, _DP), _F32),
            pltpu.VMEM((_GC, _DP), _F32),
            pltpu.VMEM((_GC, _DP), _F32),
            pltpu.SemaphoreType.DMA,
            pltpu.SemaphoreType.DMA,
        ],
    )
    def gather3(ktab, qtab, vtab, src, dst, ok, oq, ov, sv0, dv0, sv1, dv1,
                bk0, bq0, bv0, bk1, bq1, bv1, sem0, sem1):
        wid = lax.axis_index("s") * _NC + lax.axis_index("c")

        def off_of(j):
            return (wid + _NW * j) * _GC

        def issue(off, sv, dv, bk, bq, bv, sm):
            pltpu.sync_copy(src.at[pl.ds(off, _GC)], sv)
            pltpu.sync_copy(dst.at[pl.ds(off, _GC)], dv)
            pltpu.async_copy(ktab.at[sv], bk, sm)
            pltpu.async_copy(qtab.at[dv], bq, sm)
            pltpu.async_copy(vtab.at[sv], bv, sm)

        def drain_wb(off, sv, dv, bk, bq, bv, sm):
            pltpu.make_async_copy(ktab.at[sv], bk, sm).wait()
            pltpu.make_async_copy(qtab.at[dv], bq, sm).wait()
            pltpu.make_async_copy(vtab.at[sv], bv, sm).wait()
            pltpu.sync_copy(bk, ok.at[pl.ds(off, _GC)])
            pltpu.sync_copy(bq, oq.at[pl.ds(off, _GC)])
            pltpu.sync_copy(bv, ov.at[pl.ds(off, _GC)])

        issue(off_of(0), sv0, dv0, bk0, bq0, bv0, sem0)

        def body(i, carry):
            j0 = 2 * i
            j1 = j0 + 1
            issue(off_of(j1), sv1, dv1, bk1, bq1, bv1, sem1)
            drain_wb(off_of(j0), sv0, dv0, bk0, bq0, bv0, sem0)

            @pl.when(j1 + 1 < _GJ)
            def _():
                issue(off_of(j1 + 1), sv0, dv0, bk0, bq0, bv0, sem0)

            drain_wb(off_of(j1), sv1, dv1, bk1, bq1, bv1, sem1)
            return carry

        lax.fori_loop(0, _GJ // 2, body, 0)

        drain_wb(off_of(_GJ - 1), sv0, dv0, bk0, bq0, bv0, sem0)

        @pl.when(wid < _GT)
        def _():
            off = (_GJ * _NW + wid) * _GC
            issue(off, sv1, dv1, bk1, bq1, bv1, sem1)
            drain_wb(off, sv1, dv1, bk1, bq1, bv1, sem1)

    return gather3


def _scatter_build():
    mesh = plsc.VectorSubcoreMesh(core_axis_name="c", subcore_axis_name="s", num_cores=_NC, num_subcores=_NS)

    @functools.partial(
        pl.kernel,
        out_type=(
            jax.ShapeDtypeStruct((_NP, _D), _F32),
            jax.ShapeDtypeStruct((_NC * _ZP, 128), _F32),
        ),
        mesh=mesh,
        scratch_types=[
            pltpu.VMEM((_SC,), jnp.int32),
            pltpu.VMEM((_SC,), jnp.int32),
            pltpu.VMEM((_SC, 128), _F32),
            pltpu.VMEM((_SC, 128), _F32),
            pltpu.VMEM((_SC * 16,), _F32),
            pltpu.VMEM((_SC * 16,), _F32),
            pltpu.VMEM((_SC,), jnp.int32),
            pltpu.SemaphoreType.DMA,
            pltpu.SemaphoreType.DMA,
            pltpu.VMEM_SHARED((_NP, 128), _F32),
        ],
    )
    def scatter(ev, s16, dst, zrows, owv, oz, dst_v, dst_v1, evb,
                evb1, sb, sb1, dstc_v, sem0, sem1, acc):
        zb = evb1
        cid = lax.axis_index("c")
        sid = lax.axis_index("s")
        col0 = cid * 128
        node0 = cid * _ZH

        pltpu.sync_copy(zrows, acc.at[pl.ds(sid * _RPS, _RPS)])
        plsc.subcore_barrier()

        def off_of(j):
            return (sid + _NS * j) * _SC

        def issue_in(off, dv, ebuf, sm):
            pltpu.async_copy(dst.at[pl.ds(off, _SC)], dv, sm)
            pltpu.async_copy(ev.at[pl.ds(off, _SC), pl.ds(col0, 128)], ebuf,
                             sm)

        def drain_in(off, dv, ebuf, sm):
            pltpu.make_async_copy(dst.at[pl.ds(off, _SC)], dv, sm).wait()
            pltpu.make_async_copy(ev.at[pl.ds(off, _SC), pl.ds(col0, 128)],
                                  ebuf, sm).wait()

        issue_in(off_of(0), dst_v, evb, sem0)

        def body_ev(i, carry):
            j0 = 2 * i
            j1 = j0 + 1
            drain_in(off_of(j0), dst_v, evb, sem0)
            issue_in(off_of(j1), dst_v1, evb1, sem1)
            pltpu.sync_copy(evb, acc.at[dst_v], add=True)
            drain_in(off_of(j1), dst_v1, evb1, sem1)

            @pl.when(j1 + 1 < _SJ)
            def _():
                issue_in(off_of(j1 + 1), dst_v, evb, sem0)

            pltpu.sync_copy(evb1, acc.at[dst_v1], add=True)
            return carry

        lax.fori_loop(0, _SJ // 2, body_ev, 0)

        @pl.when(sid < _ST)
        def _():
            off = (_SJ * _NS + sid) * _SC
            pltpu.sync_copy(dst.at[pl.ds(off, _SC)], dst_v)
            pltpu.sync_copy(ev.at[pl.ds(off, _SC), pl.ds(col0, 128)], evb)
            pltpu.sync_copy(evb, acc.at[dst_v], add=True)
        plsc.subcore_barrier()
        r0 = sid * _RPS
        pltpu.sync_copy(acc.at[pl.ds(r0, _RPS)],
                        owv.at[pl.ds(r0, _RPS), pl.ds(col0, 128)])
        plsc.subcore_barrier()

        def zoff(j):
            return (sid + _NS * j) * _SC

        pltpu.sync_copy(zrows.at[pl.ds(0, _ZRS)],
                        acc.at[pl.ds(sid * _ZRS, _ZRS)])
        plsc.subcore_barrier()

        pltpu.sync_copy(zrows.at[pl.ds(0, _SC)], zb)

        def issue_z(off, dv, sbuf, sm):
            pltpu.async_copy(dst.at[pl.ds(off, _SC)], dv, sm)
            pltpu.async_copy(s16.at[pl.ds(off * 16, _SC * 16)], sbuf, sm)

        def drain_z(off, dv, sbuf, sm):
            pltpu.make_async_copy(dst.at[pl.ds(off, _SC)], dv, sm).wait()
            pltpu.make_async_copy(s16.at[pl.ds(off * 16, _SC * 16)], sbuf,
                                  sm).wait()

        def work_z(dv, sbuf):
            for t in range(_SC):
                zb[t, pl.ds(0, 16)] = sbuf[pl.ds(t * 16, 16)]
            for t in range(_SC // 16):
                iv = dv[pl.ds(t * 16, 16)]
                rel = iv - node0
                good = (rel >= 0) & (rel < _ZH)
                dstc_v[pl.ds(t * 16, 16)] = jnp.where(good, rel, _ZH)
            pltpu.sync_copy(zb, acc.at[dstc_v], add=True)

        issue_z(zoff(0), dst_v, sb, sem0)

        def body_z(i, carry):
            j0 = 2 * i
            j1 = j0 + 1
            issue_z(zoff(j1), dst_v1, sb1, sem1)
            drain_z(zoff(j0), dst_v, sb, sem0)
            work_z(dst_v, sb)

            @pl.when(j1 + 1 < _SJ)
            def _():
                issue_z(zoff(j1 + 1), dst_v, sb, sem0)

            drain_z(zoff(j1), dst_v1, sb1, sem1)
            work_z(dst_v1, sb1)
            return carry

        lax.fori_loop(0, _SJ // 2, body_z, 0)

        @pl.when(sid < _ST)
        def _():
            off = (_SJ * _NS + sid) * _SC
            pltpu.sync_copy(dst.at[pl.ds(off, _SC)], dst_v)
            pltpu.sync_copy(s16.at[pl.ds(off * 16, _SC * 16)], sb)
            work_z(dst_v, sb)
        plsc.subcore_barrier()
        rz = sid * _ZRS
        pltpu.sync_copy(acc.at[pl.ds(rz, _ZRS)],
                        oz.at[pl.ds(cid * _ZP + rz, _ZRS)])

    return scatter


_GATHER3 = None
_SCATTER = None


def _gather3_run(k, q, vv, src, dst):
    global _GATHER3
    if _GATHER3 is None:
        _GATHER3 = _gather3_build()
    return _GATHER3(k, q, vv, src, dst)


def _scatter_run(ev, s16, dst, zrows):
    global _SCATTER
    if _SCATTER is None:
        _SCATTER = _scatter_build()
    return _SCATTER(ev, s16, dst, zrows)


_SMASK = (np.arange(_D)[:, None] // _DH ==
          np.arange(_H)[None, :]).astype(np.float32)
_BMASK = (np.arange(_D)[None, :] // _DH ==
          np.arange(_H)[:, None]).astype(np.float32)
_BMZ = np.concatenate([_BMASK, np.zeros((8, _D), np.float32)], axis=0)


def _row(x):
    return x.reshape(1, -1)


def kernel(v, e, edge_index, WQ, WK, WV, We, WOv, bOv, WOe, bOe, W1v, b1v,
           W2v, b2v, W1e, b1e, W2e, b2e, g1v, be1v, g1e, be1e, g2v, be2v,
           g2e, be2e):
    src = edge_index[0]
    dst = edge_index[1]
    wqkv = jnp.concatenate([WQ, WK, WV], axis=1)
    we16 = We.astype(_BF16)
    woe16 = WOe.astype(_BF16)
    wov16 = WOv.astype(_BF16)

    q, k, vv = _qkv_call(v, wqkv)

    ksrc, qdst, vsrc = _gather3_run(k, q, vv, src, dst)

    e1, ev, s16, s1, q1 = _edge_a_call(e, ksrc, qdst, vsrc, we16, woe16,
                                       _row(bOe), _SMASK, _BMASK)

    zrows = jnp.zeros((_RPS, 128), _F32)
    wv_pad, oz = _scatter_run(ev, s16.reshape(-1), dst, zrows)
    wv = wv_pad[:_N]
    z16 = jnp.concatenate(
        [oz[:_ZH, :16], oz[_ZP:_ZP + _N - _ZH, :16]], axis=0)

    m1 = s1 / _E
    r1 = lax.rsqrt(q1 / _E - m1 * m1 + 1e-5)
    e2, s2, q2 = _bnffn_call(e1, m1, r1, _row(g1e), _row(be1e), W1e.astype(_BF16),
                             _row(b1e), W2e, _row(b2e), _E, 1000)
    m2 = s2 / _E
    r2 = lax.rsqrt(q2 / _E - m2 * m2 + 1e-5)
    out_e = _bnapply_call(e2, m2, r2, _row(g2e), _row(be2e), _E, 1000)

    v1, sv1, qv1 = _vatt_call(wv, z16, v, wov16, _row(bOv), _BMZ)
    mv1 = sv1 / _N
    rv1 = lax.rsqrt(qv1 / _N - mv1 * mv1 + 1e-5)
    v2, sv2, qv2 = _bnffn_call(v1, mv1, rv1, _row(g1v), _row(be1v), W1v.astype(_BF16),
                               _row(b1v), W2v.astype(_BF16), _row(b2v), _N, 400)
    mv2 = sv2 / _N
    rv2 = lax.rsqrt(qv2 / _N - mv2 * mv2 + 1e-5)
    out_v = _bnapply_call(v2, mv2, rv2, _row(g2v), _row(be2v), _N, 400)

    return (out_v, out_e)
